# Initial kernel scaffold; baseline (speedup 1.0000x reference)
import functools

import jax
import jax.numpy as jnp
from jax import lax
from jax.experimental import pallas as pl
from jax.experimental.pallas import tpu as pltpu
from jax.experimental.pallas import tpu_sc as plsc

N = 10000
E = 320000
D_IN = 128
H = 8
D_H = 16
HID = H * D_H
D_OUT = 128
NEG = 0.2

NS = 16
EPT = E // NS
CH = 80
NCHUNK = EPT // CH
PAIRS = CH // 2
NPT = N // NS
NB = 125
NNB = NPT // NB

BN = 2500


def _pre_body(x_ref, w_ref, lw_ref, feat_ref, el_ref, er_ref):
    feat = jnp.dot(x_ref[...], w_ref[0], preferred_element_type=jnp.float32)
    lg = jnp.dot(feat, lw_ref[0], preferred_element_type=jnp.float32)
    feat_ref[...] = feat[None]
    el_ref[...] = lg[:, :16][None]
    er_ref[...] = lg[:, 16:][None]


_pre_call = pl.pallas_call(
    _pre_body,
    grid=(2, N // BN),
    in_specs=[
        pl.BlockSpec((BN, D_IN), lambda m, i: (i, 0)),
        pl.BlockSpec((1, D_IN, HID), lambda m, i: (m, 0, 0)),
        pl.BlockSpec((1, HID, 32), lambda m, i: (m, 0, 0)),
    ],
    out_specs=[
        pl.BlockSpec((1, BN, HID), lambda m, i: (m, i, 0)),
        pl.BlockSpec((1, BN, 16), lambda m, i: (m, i, 0)),
        pl.BlockSpec((1, BN, 16), lambda m, i: (m, i, 0)),
    ],
    out_shape=[
        jax.ShapeDtypeStruct((2, N, HID), jnp.float32),
        jax.ShapeDtypeStruct((2, N, 16), jnp.float32),
        jax.ShapeDtypeStruct((2, N, 16), jnp.float32),
    ],
)


def _post_body(h1_ref, h2_ref, b1_ref, b2_ref, ws_ref, wfc_ref, bfc_ref,
               out_ref):
    h1 = h1_ref[...] + b1_ref[...]
    h2 = h2_ref[...] + b2_ref[...]
    ws = ws_ref[...]
    s1 = jnp.sum(h1 * ws, axis=1, keepdims=True)
    s2 = jnp.sum(h2 * ws, axis=1, keepdims=True)
    m = jnp.maximum(s1, s2)
    e1 = jnp.exp(s1 - m)
    e2 = jnp.exp(s2 - m)
    h = (e1 * h1 + e2 * h2) / (e1 + e2)
    out_ref[...] = (
        jnp.dot(h, wfc_ref[...], preferred_element_type=jnp.float32)
        + bfc_ref[...]
    )


_post_call = pl.pallas_call(
    _post_body,
    grid=(N // BN,),
    in_specs=[
        pl.BlockSpec((BN, HID), lambda i: (i, 0)),
        pl.BlockSpec((BN, HID), lambda i: (i, 0)),
        pl.BlockSpec((1, HID), lambda i: (0, 0)),
        pl.BlockSpec((1, HID), lambda i: (0, 0)),
        pl.BlockSpec((1, HID), lambda i: (0, 0)),
        pl.BlockSpec((HID, D_OUT), lambda i: (0, 0)),
        pl.BlockSpec((1, D_OUT), lambda i: (0, 0)),
    ],
    out_specs=pl.BlockSpec((BN, D_OUT), lambda i: (i, 0)),
    out_shape=jax.ShapeDtypeStruct((N, D_OUT), jnp.float32),
)


def _sc_body(feat_h, el_h, er_h, src_h, dst_h,
             h_o, alpha_o, ez_o,
             rst_sh, den_sh,
             srcv, dstv, srcav, dstav,
             elv, erv, ezv, ezpv, apv, featv,
             nbufv, dbufv):
    cid = lax.axis_index("c")
    sid = lax.axis_index("s")
    nbase = cid * N
    ebase = cid * E + sid * EPT
    zero16 = jnp.zeros((16,), jnp.float32)
    lanes = lax.iota(jnp.int32, 16)
    lmask = lanes < 8

    def zrow(j, c):
        for t in range(H):
            nbufv[j, pl.ds(t * 16, 16)] = zero16
        dbufv[j, pl.ds(0, 16)] = zero16
        return c

    lax.fori_loop(0, NB, zrow, 0)

    def zcp(k, c):
        roff = sid * NPT + k * NB
        pltpu.sync_copy(nbufv, rst_sh.at[pl.ds(roff, NB)])
        pltpu.sync_copy(dbufv, den_sh.at[pl.ds(roff, NB)])
        return c

    lax.fori_loop(0, NNB, zcp, 0)
    plsc.subcore_barrier()

    def chunk1(i, c):
        off = pl.multiple_of(ebase + i * CH, 8)
        pltpu.sync_copy(src_h.at[pl.ds(off, CH)], srcv)
        pltpu.sync_copy(dst_h.at[pl.ds(off, CH)], dstv)

        def adj(k, cc):
            s = pl.ds(k * 16, 16)
            srcav[s] = srcv[s] + nbase
            dstav[s] = dstv[s] + nbase
            return cc

        lax.fori_loop(0, CH // 16, adj, 0)
        pltpu.sync_copy(el_h.at[srcav], elv)
        pltpu.sync_copy(er_h.at[dstav], erv)

        def pair(p, cc):
            e0 = elv[2 * p] + erv[2 * p]
            e0 = jnp.where(e0 >= 0, e0, NEG * e0)
            z0 = jnp.exp(e0)
            e1 = elv[2 * p + 1] + erv[2 * p + 1]
            e1 = jnp.where(e1 >= 0, e1, NEG * e1)
            z1 = jnp.exp(e1)
            ezv[2 * p] = z0
            ezv[2 * p + 1] = z1
            ezpv[p] = jnp.where(lmask, z0, z1)
            return cc

        lax.fori_loop(0, PAIRS, pair, 0)
        pltpu.sync_copy(ezv, den_sh.at[dstv], add=True)
        poff = pl.multiple_of(off // 2, 8)
        pltpu.sync_copy(ezpv, ez_o.at[pl.ds(poff, PAIRS)])
        pltpu.sync_copy(feat_h.at[srcav], featv)

        def scale(j, cc):
            for t in range(H):
                sc = ezv[j, t]
                sl = pl.ds(t * 16, 16)
                featv[j, sl] = featv[j, sl] * sc
            return cc

        lax.fori_loop(0, CH, scale, 0)
        pltpu.sync_copy(featv, rst_sh.at[dstv], add=True)
        return c

    lax.fori_loop(0, NCHUNK, chunk1, 0)
    plsc.subcore_barrier()

    def npass(k, c):
        roff = sid * NPT + k * NB
        pltpu.sync_copy(rst_sh.at[pl.ds(roff, NB)], nbufv)
        pltpu.sync_copy(den_sh.at[pl.ds(roff, NB)], dbufv)

        def nrow(j, cc):
            for t in range(H):
                d = dbufv[j, t] + 1e-16
                sl = pl.ds(t * 16, 16)
                nbufv[j, sl] = nbufv[j, sl] / d
            return cc

        lax.fori_loop(0, NB, nrow, 0)
        pltpu.sync_copy(nbufv, h_o.at[pl.ds(nbase + roff, NB)])
        return c

    lax.fori_loop(0, NNB, npass, 0)

    def chunk2(i, c):
        off = pl.multiple_of(ebase + i * CH, 8)
        pltpu.sync_copy(dst_h.at[pl.ds(off, CH)], dstv)
        pltpu.sync_copy(den_sh.at[dstv], erv)
        poff = pl.multiple_of(off // 2, 8)
        pltpu.sync_copy(ez_o.at[pl.ds(poff, PAIRS)], ezpv)

        def pair2(p, cc):
            d0 = erv[2 * p]
            d1 = erv[2 * p + 1]
            dp = jnp.where(lmask, d0, d1) + 1e-16
            apv[p] = ezpv[p] / dp
            return cc

        lax.fori_loop(0, PAIRS, pair2, 0)
        pltpu.sync_copy(apv, alpha_o.at[pl.ds(poff, PAIRS)])
        return c

    lax.fori_loop(0, NCHUNK, chunk2, 0)


_sc_call = functools.partial(
    pl.kernel,
    out_type=(
        jax.ShapeDtypeStruct((2 * N, HID), jnp.float32),
        jax.ShapeDtypeStruct((E, 16), jnp.float32),
        jax.ShapeDtypeStruct((E, 16), jnp.float32),
    ),
    mesh=plsc.VectorSubcoreMesh(core_axis_name="c", subcore_axis_name="s"),
    scratch_types=[
        pltpu.VMEM_SHARED((N, HID), jnp.float32),
        pltpu.VMEM_SHARED((N, 16), jnp.float32),
        pltpu.VMEM((CH,), jnp.int32),
        pltpu.VMEM((CH,), jnp.int32),
        pltpu.VMEM((CH,), jnp.int32),
        pltpu.VMEM((CH,), jnp.int32),
        pltpu.VMEM((CH, 16), jnp.float32),
        pltpu.VMEM((CH, 16), jnp.float32),
        pltpu.VMEM((CH, 16), jnp.float32),
        pltpu.VMEM((PAIRS, 16), jnp.float32),
        pltpu.VMEM((PAIRS, 16), jnp.float32),
        pltpu.VMEM((CH, HID), jnp.float32),
        pltpu.VMEM((NB, HID), jnp.float32),
        pltpu.VMEM((NB, 16), jnp.float32),
    ],
)(_sc_body)


def _mk_diag(a):
    rows = jnp.arange(HID)
    cols = rows // D_H
    z = jnp.zeros((HID, H), jnp.float32)
    return z.at[rows, cols].set(a.reshape(-1))


def kernel(x, edge_index_pap, edge_index_pfp, W_pap, al_pap, ar_pap, b_pap,
           W_pfp, al_pfp, ar_pfp, b_pfp, W_sem, b_sem, W_fc, b_fc):
    ALp, ARp = _mk_diag(al_pap), _mk_diag(ar_pap)
    ALf, ARf = _mk_diag(al_pfp), _mk_diag(ar_pfp)
    LW = jnp.stack([
        jnp.concatenate([ALp, ALp, ARp, ARp], axis=1),
        jnp.concatenate([ALf, ALf, ARf, ARf], axis=1),
    ])
    W_s = jnp.stack([W_pap, W_pfp])

    feat_s, el_s, er_s = _pre_call(x, W_s, LW)
    src = jnp.concatenate([edge_index_pap[0], edge_index_pfp[0]])
    dst = jnp.concatenate([edge_index_pap[1], edge_index_pfp[1]])

    h_o, alpha_o, _ = _sc_call(
        feat_s.reshape(2 * N, HID),
        el_s.reshape(2 * N, 16),
        er_s.reshape(2 * N, 16),
        src,
        dst,
    )
    h_all = h_o.reshape(2, N, HID)
    out = _post_call(
        h_all[0], h_all[1],
        b_pap.reshape(1, HID), b_pfp.reshape(1, HID),
        W_sem.reshape(1, HID),
        W_fc, b_fc.reshape(1, D_OUT),
    )
    a1 = alpha_o[: E // 2].reshape(E, H, 1)
    a2 = alpha_o[E // 2:].reshape(E, H, 1)
    return out, a1, a2


# speedup vs baseline: 34.9832x; 34.9832x over previous
import functools

import jax
import jax.numpy as jnp
from jax import lax
from jax.experimental import pallas as pl
from jax.experimental.pallas import tpu as pltpu
from jax.experimental.pallas import tpu_sc as plsc

N = 10000
E = 320000
D_IN = 128
H = 8
D_H = 16
HID = H * D_H
D_OUT = 128
NEG = 0.2

NS = 16
EPT = E // NS
CH = 80
NCHUNK = EPT // CH
PAIRS = CH // 2
NPAD = 10240
NPT = NPAD // NS
NB = 128
NNB = NPT // NB

BN = 2000


def _pre_body(x_ref, w_ref, lw_ref, feat_ref, el_ref, er_ref):
    feat = jnp.dot(x_ref[...], w_ref[0], preferred_element_type=jnp.float32)
    lg = jnp.dot(feat, lw_ref[0], preferred_element_type=jnp.float32)
    feat_ref[...] = feat[None]
    el_ref[...] = lg[:, :16][None]
    er_ref[...] = lg[:, 16:][None]


_pre_call = pl.pallas_call(
    _pre_body,
    grid=(2, N // BN),
    in_specs=[
        pl.BlockSpec((BN, D_IN), lambda m, i: (i, 0)),
        pl.BlockSpec((1, D_IN, HID), lambda m, i: (m, 0, 0)),
        pl.BlockSpec((1, HID, 32), lambda m, i: (m, 0, 0)),
    ],
    out_specs=[
        pl.BlockSpec((1, BN, HID), lambda m, i: (m, i, 0)),
        pl.BlockSpec((1, BN, 16), lambda m, i: (m, i, 0)),
        pl.BlockSpec((1, BN, 16), lambda m, i: (m, i, 0)),
    ],
    out_shape=[
        jax.ShapeDtypeStruct((2, N, HID), jnp.float32),
        jax.ShapeDtypeStruct((2, N, 16), jnp.float32),
        jax.ShapeDtypeStruct((2, N, 16), jnp.float32),
    ],
)


def _post_body(h1_ref, h2_ref, b1_ref, b2_ref, ws_ref, wfc_ref, bfc_ref,
               out_ref):
    h1 = h1_ref[...] + b1_ref[...]
    h2 = h2_ref[...] + b2_ref[...]
    ws = ws_ref[...]
    s1 = jnp.sum(h1 * ws, axis=1, keepdims=True)
    s2 = jnp.sum(h2 * ws, axis=1, keepdims=True)
    m = jnp.maximum(s1, s2)
    e1 = jnp.exp(s1 - m)
    e2 = jnp.exp(s2 - m)
    h = (e1 * h1 + e2 * h2) / (e1 + e2)
    out_ref[...] = (
        jnp.dot(h, wfc_ref[...], preferred_element_type=jnp.float32)
        + bfc_ref[...]
    )


_post_call = pl.pallas_call(
    _post_body,
    grid=(N // BN,),
    in_specs=[
        pl.BlockSpec((BN, HID), lambda i: (i, 0)),
        pl.BlockSpec((BN, HID), lambda i: (i, 0)),
        pl.BlockSpec((1, HID), lambda i: (0, 0)),
        pl.BlockSpec((1, HID), lambda i: (0, 0)),
        pl.BlockSpec((1, HID), lambda i: (0, 0)),
        pl.BlockSpec((HID, D_OUT), lambda i: (0, 0)),
        pl.BlockSpec((1, D_OUT), lambda i: (0, 0)),
    ],
    out_specs=pl.BlockSpec((BN, D_OUT), lambda i: (i, 0)),
    out_shape=jax.ShapeDtypeStruct((N, D_OUT), jnp.float32),
)


def _sc_body(feat_h, el_h, er_h, src_h, dst_h,
             h_o, alpha_o, ez_o,
             rst_sh, den_sh,
             srcv, dstv, srcav, dstav,
             elv, erv, ezv, ezpv, apv, featv,
             nbufv, dbufv):
    cid = lax.axis_index("c")
    sid = lax.axis_index("s")
    nbase = cid * N
    abase = cid * NPAD
    ebase = cid * E + sid * EPT
    zero16 = jnp.zeros((16,), jnp.float32)
    lanes = lax.iota(jnp.int32, 16)
    lmask = lanes < 8

    def zrow(j, c):
        for t in range(H):
            nbufv[j, pl.ds(t * 16, 16)] = zero16
        dbufv[j, pl.ds(0, 16)] = zero16
        return c

    lax.fori_loop(0, NB, zrow, 0)

    def zcp(k, c):
        roff = sid * NPT + k * NB
        pltpu.sync_copy(nbufv, rst_sh.at[pl.ds(roff, NB)])
        pltpu.sync_copy(dbufv, den_sh.at[pl.ds(roff, NB)])
        return c

    lax.fori_loop(0, NNB, zcp, 0)
    plsc.subcore_barrier()

    def chunk1(i, c):
        off = pl.multiple_of(ebase + i * CH, 8)
        pltpu.sync_copy(src_h.at[pl.ds(off, CH)], srcv)
        pltpu.sync_copy(dst_h.at[pl.ds(off, CH)], dstv)

        def adj(k, cc):
            s = pl.ds(k * 16, 16)
            srcav[s] = srcv[s] + nbase
            dstav[s] = dstv[s] + nbase
            return cc

        lax.fori_loop(0, CH // 16, adj, 0)
        pltpu.sync_copy(el_h.at[srcav], elv)
        pltpu.sync_copy(er_h.at[dstav], erv)

        def pair(p, cc):
            e0 = elv[2 * p] + erv[2 * p]
            e0 = jnp.where(e0 >= 0, e0, NEG * e0)
            z0 = jnp.exp(e0)
            e1 = elv[2 * p + 1] + erv[2 * p + 1]
            e1 = jnp.where(e1 >= 0, e1, NEG * e1)
            z1 = jnp.exp(e1)
            ezv[2 * p] = z0
            ezv[2 * p + 1] = z1
            ezpv[p] = jnp.where(lmask, z0, z1)
            return cc

        lax.fori_loop(0, PAIRS, pair, 0)
        pltpu.sync_copy(ezv, den_sh.at[dstv], add=True)
        poff = pl.multiple_of(off // 2, 8)
        pltpu.sync_copy(ezpv, ez_o.at[pl.ds(poff, PAIRS)])
        pltpu.sync_copy(feat_h.at[srcav], featv)

        def scale(j, cc):
            ez_row = ezv[j]
            for t in range(H):
                sl = pl.ds(t * 16, 16)
                featv[j, sl] = featv[j, sl] * ez_row[t]
            return cc

        lax.fori_loop(0, CH, scale, 0)
        pltpu.sync_copy(featv, rst_sh.at[dstv], add=True)
        return c

    lax.fori_loop(0, NCHUNK, chunk1, 0)
    plsc.subcore_barrier()

    def npass(k, c):
        roff = sid * NPT + k * NB
        pltpu.sync_copy(rst_sh.at[pl.ds(roff, NB)], nbufv)
        pltpu.sync_copy(den_sh.at[pl.ds(roff, NB)], dbufv)

        def nrow(j, cc):
            d_row = dbufv[j]
            for t in range(H):
                sl = pl.ds(t * 16, 16)
                nbufv[j, sl] = nbufv[j, sl] / (d_row[t] + 1e-16)
            return cc

        lax.fori_loop(0, NB, nrow, 0)
        pltpu.sync_copy(nbufv, h_o.at[pl.ds(abase + roff, NB)])
        return c

    lax.fori_loop(0, NNB, npass, 0)

    def chunk2(i, c):
        off = pl.multiple_of(ebase + i * CH, 8)
        pltpu.sync_copy(dst_h.at[pl.ds(off, CH)], dstv)
        pltpu.sync_copy(den_sh.at[dstv], erv)
        poff = pl.multiple_of(off // 2, 8)
        pltpu.sync_copy(ez_o.at[pl.ds(poff, PAIRS)], ezpv)

        def pair2(p, cc):
            d0 = erv[2 * p]
            d1 = erv[2 * p + 1]
            dp = jnp.where(lmask, d0, d1) + 1e-16
            apv[p] = ezpv[p] / dp
            return cc

        lax.fori_loop(0, PAIRS, pair2, 0)
        pltpu.sync_copy(apv, alpha_o.at[pl.ds(poff, PAIRS)])
        return c

    lax.fori_loop(0, NCHUNK, chunk2, 0)


@functools.cache
def _get_sc_call():
    return pl.kernel(
        _sc_body,
        out_type=(
            jax.ShapeDtypeStruct((2 * NPAD, HID), jnp.float32),
            jax.ShapeDtypeStruct((E, 16), jnp.float32),
            jax.ShapeDtypeStruct((E, 16), jnp.float32),
        ),
        mesh=plsc.VectorSubcoreMesh(core_axis_name="c", subcore_axis_name="s",
                                    num_cores=2, num_subcores=NS),
        compiler_params=pltpu.CompilerParams(use_tc_tiling_on_sc=False),
        scratch_types=[
        pltpu.VMEM_SHARED((NPAD, HID), jnp.float32),
        pltpu.VMEM_SHARED((NPAD, 16), jnp.float32),
        pltpu.VMEM((CH,), jnp.int32),
        pltpu.VMEM((CH,), jnp.int32),
        pltpu.VMEM((CH,), jnp.int32),
        pltpu.VMEM((CH,), jnp.int32),
        pltpu.VMEM((CH, 16), jnp.float32),
        pltpu.VMEM((CH, 16), jnp.float32),
        pltpu.VMEM((CH, 16), jnp.float32),
        pltpu.VMEM((PAIRS, 16), jnp.float32),
        pltpu.VMEM((PAIRS, 16), jnp.float32),
            pltpu.VMEM((CH, HID), jnp.float32),
            pltpu.VMEM((NB, HID), jnp.float32),
            pltpu.VMEM((NB, 16), jnp.float32),
        ],
    )


def _mk_diag(a):
    rows = jnp.arange(HID)
    cols = rows // D_H
    z = jnp.zeros((HID, H), jnp.float32)
    return z.at[rows, cols].set(a.reshape(-1))


def kernel(x, edge_index_pap, edge_index_pfp, W_pap, al_pap, ar_pap, b_pap,
           W_pfp, al_pfp, ar_pfp, b_pfp, W_sem, b_sem, W_fc, b_fc):
    ALp, ARp = _mk_diag(al_pap), _mk_diag(ar_pap)
    ALf, ARf = _mk_diag(al_pfp), _mk_diag(ar_pfp)
    LW = jnp.stack([
        jnp.concatenate([ALp, ALp, ARp, ARp], axis=1),
        jnp.concatenate([ALf, ALf, ARf, ARf], axis=1),
    ])
    W_s = jnp.stack([W_pap, W_pfp])

    feat_s, el_s, er_s = _pre_call(x, W_s, LW)
    src = jnp.concatenate([edge_index_pap[0], edge_index_pfp[0]])
    dst = jnp.concatenate([edge_index_pap[1], edge_index_pfp[1]])

    h_o, alpha_o, _ = _get_sc_call()(
        feat_s.reshape(2 * N, HID),
        el_s.reshape(2 * N, 16),
        er_s.reshape(2 * N, 16),
        src,
        dst,
    )
    h_all = h_o.reshape(2, NPAD, HID)[:, :N, :]
    out = _post_call(
        h_all[0], h_all[1],
        b_pap.reshape(1, HID), b_pfp.reshape(1, HID),
        W_sem.reshape(1, HID),
        W_fc, b_fc.reshape(1, D_OUT),
    )
    a1 = alpha_o[: E // 2].reshape(E, H, 1)
    a2 = alpha_o[E // 2:].reshape(E, H, 1)
    return out, a1, a2


# speedup vs baseline: 63.2115x; 1.8069x over previous
import functools

import jax
import jax.numpy as jnp
from jax import lax
from jax.experimental import pallas as pl
from jax.experimental.pallas import tpu as pltpu
from jax.experimental.pallas import tpu_sc as plsc

N = 10000
E = 320000
D_IN = 128
H = 8
D_H = 16
HID = H * D_H
D_OUT = 128
NEG = 0.2

NS = 16
EPT = E // NS
CH = 80
NCHUNK = EPT // CH
PAIRS = CH // 2
NPAD = 10240
NPT = NPAD // NS
NB = 64
NNB = NPT // NB
MAC = 400
NMC = EPT // MAC
CPM = MAC // CH
MPAIRS = MAC // 2

BN = 2000


def _pre_body(x_ref, w_ref, lw_ref, feat_ref, el_ref, er_ref):
    feat = jnp.dot(x_ref[...], w_ref[0], preferred_element_type=jnp.float32)
    lg = jnp.dot(feat, lw_ref[0], preferred_element_type=jnp.float32)
    feat_ref[...] = feat[None]
    el_ref[...] = lg[:, :16][None]
    er_ref[...] = lg[:, 16:][None]


_pre_call = pl.pallas_call(
    _pre_body,
    grid=(2, N // BN),
    in_specs=[
        pl.BlockSpec((BN, D_IN), lambda m, i: (i, 0)),
        pl.BlockSpec((1, D_IN, HID), lambda m, i: (m, 0, 0)),
        pl.BlockSpec((1, HID, 32), lambda m, i: (m, 0, 0)),
    ],
    out_specs=[
        pl.BlockSpec((1, BN, HID), lambda m, i: (m, i, 0)),
        pl.BlockSpec((1, BN, 16), lambda m, i: (m, i, 0)),
        pl.BlockSpec((1, BN, 16), lambda m, i: (m, i, 0)),
    ],
    out_shape=[
        jax.ShapeDtypeStruct((2, N, HID), jnp.float32),
        jax.ShapeDtypeStruct((2, N, 16), jnp.float32),
        jax.ShapeDtypeStruct((2, N, 16), jnp.float32),
    ],
)


def _post_body(h1_ref, h2_ref, b1_ref, b2_ref, ws_ref, wfc_ref, bfc_ref,
               out_ref):
    h1 = h1_ref[0] + b1_ref[...]
    h2 = h2_ref[0] + b2_ref[...]
    ws = ws_ref[...]
    s1 = jnp.sum(h1 * ws, axis=1, keepdims=True)
    s2 = jnp.sum(h2 * ws, axis=1, keepdims=True)
    m = jnp.maximum(s1, s2)
    e1 = jnp.exp(s1 - m)
    e2 = jnp.exp(s2 - m)
    h = (e1 * h1 + e2 * h2) / (e1 + e2)
    out_ref[...] = (
        jnp.dot(h, wfc_ref[...], preferred_element_type=jnp.float32)
        + bfc_ref[...]
    )


_post_call = pl.pallas_call(
    _post_body,
    grid=(N // BN,),
    in_specs=[
        pl.BlockSpec((1, BN, HID), lambda i: (0, i, 0)),
        pl.BlockSpec((1, BN, HID), lambda i: (1, i, 0)),
        pl.BlockSpec((1, HID), lambda i: (0, 0)),
        pl.BlockSpec((1, HID), lambda i: (0, 0)),
        pl.BlockSpec((1, HID), lambda i: (0, 0)),
        pl.BlockSpec((HID, D_OUT), lambda i: (0, 0)),
        pl.BlockSpec((1, D_OUT), lambda i: (0, 0)),
    ],
    out_specs=pl.BlockSpec((BN, D_OUT), lambda i: (i, 0)),
    out_shape=jax.ShapeDtypeStruct((N, D_OUT), jnp.float32),
)


def _sc_body(feat_h, el_h, er_h, src_h, dst_h,
             h_o, alpha1_o, alpha2_o, ez_o,
             rst_sh, den_sh,
             srcb, dstb, dstab, srcscv, dstscv, dstav,
             elv, erv, ezv, ezpb, apb, featv,
             nbufv, dbufv,
             semi, sema, semb, semc, semd, sems):
    cid = lax.axis_index("c")
    sid = lax.axis_index("s")
    nbase = cid * N
    ebase = cid * E + sid * EPT
    zero16 = jnp.zeros((16,), jnp.float32)
    lanes = lax.iota(jnp.int32, 16)
    lmask = lanes < 8

    def zrow(j, c):
        for t in range(H):
            nbufv[j, pl.ds(t * 16, 16)] = zero16
        dbufv[j, pl.ds(0, 16)] = zero16
        return c

    lax.fori_loop(0, NB, zrow, 0)

    def zcp(k, c):
        roff = sid * NPT + k * NB
        pltpu.sync_copy(nbufv, rst_sh.at[pl.ds(roff, NB)])
        pltpu.sync_copy(dbufv, den_sh.at[pl.ds(roff, NB)])
        return c

    lax.fori_loop(0, NNB, zcp, 0)
    plsc.subcore_barrier()

    def macro1(m, c):
        moff = pl.multiple_of(ebase + m * MAC, 8)
        ld_s = pltpu.async_copy(src_h.at[pl.ds(moff, MAC)], srcb, semi)
        ld_d = pltpu.async_copy(dst_h.at[pl.ds(moff, MAC)], dstb, semi)
        ld_s.wait()
        ld_d.wait()

        def adj(k, cc):
            s = pl.ds(k * 16, 16)
            srcb[s] = srcb[s] + nbase
            dstab[s] = dstb[s] + nbase
            return cc

        lax.fori_loop(0, MAC // 16, adj, 0)

        def chunk1(j, cc):
            def cpi(k, cc2):
                s = pl.ds(k * 16, 16)
                t = pl.ds(j * CH + k * 16, 16)
                srcscv[s] = srcb[t]
                dstscv[s] = dstb[t]
                dstav[s] = dstab[t]
                return cc2

            lax.fori_loop(0, CH // 16, cpi, 0)
            g_el = pltpu.async_copy(el_h.at[srcscv], elv, sema)
            g_er = pltpu.async_copy(er_h.at[dstav], erv, semb)
            g_f = pltpu.async_copy(feat_h.at[srcscv], featv, semc)
            g_el.wait()
            g_er.wait()

            def pair(p, cc2):
                e0 = elv[2 * p] + erv[2 * p]
                e0 = jnp.where(e0 >= 0, e0, NEG * e0)
                z0 = jnp.exp(e0)
                e1 = elv[2 * p + 1] + erv[2 * p + 1]
                e1 = jnp.where(e1 >= 0, e1, NEG * e1)
                z1 = jnp.exp(e1)
                ezv[2 * p] = z0
                ezv[2 * p + 1] = z1
                ezpb[j * PAIRS + p] = jnp.where(lmask, z0, z1)
                return cc2

            lax.fori_loop(0, PAIRS, pair, 0)
            d_den = pltpu.async_copy(ezv, den_sh.at[dstscv], semd, add=True)
            g_f.wait()

            def scale(q, cc2):
                ez_row = ezv[q]
                for t in range(H):
                    sl = pl.ds(t * 16, 16)
                    featv[q, sl] = featv[q, sl] * ez_row[t]
                return cc2

            lax.fori_loop(0, CH, scale, 0)
            d_den.wait()
            pltpu.sync_copy(featv, rst_sh.at[dstscv], add=True)
            return cc

        lax.fori_loop(0, CPM, chunk1, 0)
        poff = pl.multiple_of((ebase + m * MAC) // 2, 8)
        pltpu.sync_copy(ezpb, ez_o.at[pl.ds(poff, MPAIRS)])
        return c

    lax.fori_loop(0, NMC, macro1, 0)
    plsc.subcore_barrier()

    def npass(k, c):
        roff = sid * NPT + k * NB
        pltpu.sync_copy(rst_sh.at[pl.ds(roff, NB)], nbufv)
        pltpu.sync_copy(den_sh.at[pl.ds(roff, NB)], dbufv)

        def nrow(j, cc):
            d_row = dbufv[j]
            for t in range(H):
                sl = pl.ds(t * 16, 16)
                nbufv[j, sl] = nbufv[j, sl] / (d_row[t] + 1e-16)
            return cc

        lax.fori_loop(0, NB, nrow, 0)
        pltpu.sync_copy(nbufv, h_o.at[cid, pl.ds(roff, NB)])
        return c

    lax.fori_loop(0, NNB, npass, 0)

    def macro2(m, c):
        moff = pl.multiple_of(ebase + m * MAC, 8)
        poff = pl.multiple_of((ebase + m * MAC) // 2, 8)
        ld_d = pltpu.async_copy(dst_h.at[pl.ds(moff, MAC)], dstb, semi)
        ld_z = pltpu.async_copy(ez_o.at[pl.ds(poff, MPAIRS)], ezpb, semb)
        ld_d.wait()

        def chunk2(j, cc):
            def cpi2(k, cc2):
                dstscv[pl.ds(k * 16, 16)] = dstb[pl.ds(j * CH + k * 16, 16)]
                return cc2

            lax.fori_loop(0, CH // 16, cpi2, 0)
            g_d = pltpu.async_copy(den_sh.at[dstscv], erv, sema)
            g_d.wait()

            def pair2(p, cc2):
                d0 = erv[2 * p]
                d1 = erv[2 * p + 1]
                dp = jnp.where(lmask, d0, d1) + 1e-16
                r = j * PAIRS + p
                apb[r] = ezpb[r] / dp
                return cc2

            lax.fori_loop(0, PAIRS, pair2, 0)
            return cc

        ld_z.wait()
        lax.fori_loop(0, CPM, chunk2, 0)
        lpoff = pl.multiple_of(sid * (EPT // 2) + m * MPAIRS, 8)

        @pl.when(cid == 0)
        def _():
            pltpu.sync_copy(apb, alpha1_o.at[pl.ds(lpoff, MPAIRS)])

        @pl.when(cid == 1)
        def _():
            pltpu.sync_copy(apb, alpha2_o.at[pl.ds(lpoff, MPAIRS)])

        return c

    lax.fori_loop(0, NMC, macro2, 0)


@functools.cache
def _get_sc_call():
    return pl.kernel(
        _sc_body,
        out_type=(
            jax.ShapeDtypeStruct((2, NPAD, HID), jnp.float32),
            jax.ShapeDtypeStruct((E // 2, 16), jnp.float32),
            jax.ShapeDtypeStruct((E // 2, 16), jnp.float32),
            jax.ShapeDtypeStruct((E, 16), jnp.float32),
        ),
        mesh=plsc.VectorSubcoreMesh(core_axis_name="c", subcore_axis_name="s",
                                    num_cores=2, num_subcores=NS),
        compiler_params=pltpu.CompilerParams(use_tc_tiling_on_sc=False),
        scratch_types=[
            pltpu.VMEM_SHARED((NPAD, HID), jnp.float32),
            pltpu.VMEM_SHARED((NPAD, 16), jnp.float32),
            pltpu.VMEM((MAC,), jnp.int32),
            pltpu.VMEM((MAC,), jnp.int32),
            pltpu.VMEM((MAC,), jnp.int32),
            pltpu.VMEM((CH,), jnp.int32),
            pltpu.VMEM((CH,), jnp.int32),
            pltpu.VMEM((CH,), jnp.int32),
            pltpu.VMEM((CH, 16), jnp.float32),
            pltpu.VMEM((CH, 16), jnp.float32),
            pltpu.VMEM((CH, 16), jnp.float32),
            pltpu.VMEM((MPAIRS, 16), jnp.float32),
            pltpu.VMEM((MPAIRS, 16), jnp.float32),
            pltpu.VMEM((CH, HID), jnp.float32),
            pltpu.VMEM((NB, HID), jnp.float32),
            pltpu.VMEM((NB, 16), jnp.float32),
            pltpu.SemaphoreType.DMA,
            pltpu.SemaphoreType.DMA,
            pltpu.SemaphoreType.DMA,
            pltpu.SemaphoreType.DMA,
            pltpu.SemaphoreType.DMA,
            pltpu.SemaphoreType.DMA,
        ],
    )


def _mk_diag(a):
    rows = jnp.arange(HID)
    cols = rows // D_H
    z = jnp.zeros((HID, H), jnp.float32)
    return z.at[rows, cols].set(a.reshape(-1))


def kernel(x, edge_index_pap, edge_index_pfp, W_pap, al_pap, ar_pap, b_pap,
           W_pfp, al_pfp, ar_pfp, b_pfp, W_sem, b_sem, W_fc, b_fc):
    ALp, ARp = _mk_diag(al_pap), _mk_diag(ar_pap)
    ALf, ARf = _mk_diag(al_pfp), _mk_diag(ar_pfp)
    LW = jnp.stack([
        jnp.concatenate([ALp, ALp, ARp, ARp], axis=1),
        jnp.concatenate([ALf, ALf, ARf, ARf], axis=1),
    ])
    W_s = jnp.stack([W_pap, W_pfp])

    feat_s, el_s, er_s = _pre_call(x, W_s, LW)
    src = jnp.concatenate([edge_index_pap[0], edge_index_pfp[0]])
    dst = jnp.concatenate([edge_index_pap[1], edge_index_pfp[1]])

    h_o, alpha1_o, alpha2_o, _ = _get_sc_call()(
        feat_s.reshape(2 * N, HID),
        el_s.reshape(2 * N, 16),
        er_s.reshape(2 * N, 16),
        src,
        dst,
    )
    out = _post_call(
        h_o, h_o,
        b_pap.reshape(1, HID), b_pfp.reshape(1, HID),
        W_sem.reshape(1, HID),
        W_fc, b_fc.reshape(1, D_OUT),
    )
    a1 = alpha1_o.reshape(E, H, 1)
    a2 = alpha2_o.reshape(E, H, 1)
    return out, a1, a2


# speedup vs baseline: 65.1363x; 1.0305x over previous
import functools

import jax
import jax.numpy as jnp
from jax import lax
from jax.experimental import pallas as pl
from jax.experimental.pallas import tpu as pltpu
from jax.experimental.pallas import tpu_sc as plsc

N = 10000
E = 320000
D_IN = 128
H = 8
D_H = 16
HID = H * D_H
D_OUT = 128
NEG = 0.2

NS = 16
EPT = E // NS
CH = 80
NCHUNK = EPT // CH
PAIRS = CH // 2
NPAD = 10240
NPT = NPAD // NS
NB = 64
NNB = NPT // NB
MAC = 400
NMC = EPT // MAC
CPM = MAC // CH
MPAIRS = MAC // 2

BN = 2000


def _pre_body(x_ref, w_ref, lw_ref, feat_ref, el_ref, er_ref):
    feat = jnp.dot(x_ref[...], w_ref[0], preferred_element_type=jnp.float32)
    lg = jnp.dot(feat, lw_ref[0], preferred_element_type=jnp.float32)
    feat_ref[...] = feat[None]
    el_ref[...] = lg[:, :16][None]
    er_ref[...] = lg[:, 16:][None]


_pre_call = pl.pallas_call(
    _pre_body,
    grid=(2, N // BN),
    in_specs=[
        pl.BlockSpec((BN, D_IN), lambda m, i: (i, 0)),
        pl.BlockSpec((1, D_IN, HID), lambda m, i: (m, 0, 0)),
        pl.BlockSpec((1, HID, 32), lambda m, i: (m, 0, 0)),
    ],
    out_specs=[
        pl.BlockSpec((1, BN, HID), lambda m, i: (m, i, 0)),
        pl.BlockSpec((1, BN, 16), lambda m, i: (m, i, 0)),
        pl.BlockSpec((1, BN, 16), lambda m, i: (m, i, 0)),
    ],
    out_shape=[
        jax.ShapeDtypeStruct((2, N, HID), jnp.float32),
        jax.ShapeDtypeStruct((2, N, 16), jnp.float32),
        jax.ShapeDtypeStruct((2, N, 16), jnp.float32),
    ],
)


def _post_body(h1_ref, h2_ref, b1_ref, b2_ref, ws_ref, wfc_ref, bfc_ref,
               out_ref):
    h1 = h1_ref[0] + b1_ref[...]
    h2 = h2_ref[0] + b2_ref[...]
    ws = ws_ref[...]
    s1 = jnp.sum(h1 * ws, axis=1, keepdims=True)
    s2 = jnp.sum(h2 * ws, axis=1, keepdims=True)
    m = jnp.maximum(s1, s2)
    e1 = jnp.exp(s1 - m)
    e2 = jnp.exp(s2 - m)
    h = (e1 * h1 + e2 * h2) / (e1 + e2)
    out_ref[...] = (
        jnp.dot(h, wfc_ref[...], preferred_element_type=jnp.float32)
        + bfc_ref[...]
    )


_post_call = pl.pallas_call(
    _post_body,
    grid=(N // BN,),
    in_specs=[
        pl.BlockSpec((1, BN, HID), lambda i: (0, i, 0)),
        pl.BlockSpec((1, BN, HID), lambda i: (1, i, 0)),
        pl.BlockSpec((1, HID), lambda i: (0, 0)),
        pl.BlockSpec((1, HID), lambda i: (0, 0)),
        pl.BlockSpec((1, HID), lambda i: (0, 0)),
        pl.BlockSpec((HID, D_OUT), lambda i: (0, 0)),
        pl.BlockSpec((1, D_OUT), lambda i: (0, 0)),
    ],
    out_specs=pl.BlockSpec((BN, D_OUT), lambda i: (i, 0)),
    out_shape=jax.ShapeDtypeStruct((N, D_OUT), jnp.float32),
)


def _sc_body(feat_h, el_h, er_h, ei1_h, ei2_h,
             h_o, alpha1_o, alpha2_o, ez_o,
             rst_sh, den_sh,
             srcb, dstb, dstab, srcscv, dstscv, dstav,
             elv, erv, ezv, ezpb, apb, featv,
             nbufv, dbufv,
             semi, sema, semb, semc, semd, sems):
    cid = lax.axis_index("c")
    sid = lax.axis_index("s")
    nbase = cid * N
    ebase = cid * E + sid * EPT
    zero16 = jnp.zeros((16,), jnp.float32)
    lanes = lax.iota(jnp.int32, 16)
    lmask = lanes < 8

    def zrow(j, c):
        for t in range(H):
            nbufv[j, pl.ds(t * 16, 16)] = zero16
        dbufv[j, pl.ds(0, 16)] = zero16
        return c

    lax.fori_loop(0, NB, zrow, 0)

    def zcp(k, c):
        roff = sid * NPT + k * NB
        pltpu.sync_copy(nbufv, rst_sh.at[pl.ds(roff, NB)])
        pltpu.sync_copy(dbufv, den_sh.at[pl.ds(roff, NB)])
        return c

    lax.fori_loop(0, NNB, zcp, 0)
    plsc.subcore_barrier()

    def macro1(m, c):
        loff = pl.multiple_of(sid * EPT + m * MAC, 8)

        @pl.when(cid == 0)
        def _():
            ld_s = pltpu.async_copy(ei1_h.at[0, pl.ds(loff, MAC)], srcb, semi)
            ld_d = pltpu.async_copy(ei1_h.at[1, pl.ds(loff, MAC)], dstb, semi)
            ld_s.wait()
            ld_d.wait()

        @pl.when(cid == 1)
        def _():
            ld_s = pltpu.async_copy(ei2_h.at[0, pl.ds(loff, MAC)], srcb, semi)
            ld_d = pltpu.async_copy(ei2_h.at[1, pl.ds(loff, MAC)], dstb, semi)
            ld_s.wait()
            ld_d.wait()

        def adj(k, cc):
            s = pl.ds(k * 16, 16)
            srcb[s] = srcb[s] + nbase
            dstab[s] = dstb[s] + nbase
            return cc

        lax.fori_loop(0, MAC // 16, adj, 0)

        def chunk1(j, cc):
            @pl.when(m * CPM + j > 0)
            def _():
                pltpu.make_async_copy(featv, rst_sh.at[dstscv], sems).wait()

            def cpi(k, cc2):
                s = pl.ds(k * 16, 16)
                t = pl.ds(j * CH + k * 16, 16)
                srcscv[s] = srcb[t]
                dstscv[s] = dstb[t]
                dstav[s] = dstab[t]
                return cc2

            lax.fori_loop(0, CH // 16, cpi, 0)
            g_el = pltpu.async_copy(el_h.at[srcscv], elv, sema)
            g_er = pltpu.async_copy(er_h.at[dstav], erv, semb)
            g_f = pltpu.async_copy(feat_h.at[srcscv], featv, semc)
            g_el.wait()
            g_er.wait()

            def pair(p, cc2):
                e0 = elv[2 * p] + erv[2 * p]
                e0 = jnp.where(e0 >= 0, e0, NEG * e0)
                z0 = jnp.exp(e0)
                e1 = elv[2 * p + 1] + erv[2 * p + 1]
                e1 = jnp.where(e1 >= 0, e1, NEG * e1)
                z1 = jnp.exp(e1)
                ezv[2 * p] = z0
                ezv[2 * p + 1] = z1
                ezpb[j * PAIRS + p] = jnp.where(lmask, z0, z1)
                return cc2

            lax.fori_loop(0, PAIRS, pair, 0)
            d_den = pltpu.async_copy(ezv, den_sh.at[dstscv], semd, add=True)
            g_f.wait()

            def scale(q, cc2):
                ez_row = ezv[q]
                for t in range(H):
                    sl = pl.ds(t * 16, 16)
                    featv[q, sl] = featv[q, sl] * ez_row[t]
                return cc2

            lax.fori_loop(0, CH, scale, 0)
            d_den.wait()
            pltpu.async_copy(featv, rst_sh.at[dstscv], sems,
                             add=True)
            return cc

        lax.fori_loop(0, CPM, chunk1, 0)
        poff = pl.multiple_of((ebase + m * MAC) // 2, 8)
        pltpu.sync_copy(ezpb, ez_o.at[pl.ds(poff, MPAIRS)])
        return c

    lax.fori_loop(0, NMC, macro1, 0)
    pltpu.make_async_copy(featv, rst_sh.at[dstscv], sems).wait()
    plsc.subcore_barrier()

    def npass(k, c):
        roff = sid * NPT + k * NB
        pltpu.sync_copy(rst_sh.at[pl.ds(roff, NB)], nbufv)
        pltpu.sync_copy(den_sh.at[pl.ds(roff, NB)], dbufv)

        def nrow(j, cc):
            d_row = dbufv[j]
            for t in range(H):
                sl = pl.ds(t * 16, 16)
                nbufv[j, sl] = nbufv[j, sl] / (d_row[t] + 1e-16)
            return cc

        lax.fori_loop(0, NB, nrow, 0)
        pltpu.sync_copy(nbufv, h_o.at[cid, pl.ds(roff, NB)])
        return c

    lax.fori_loop(0, NNB, npass, 0)

    def macro2(m, c):
        loff = pl.multiple_of(sid * EPT + m * MAC, 8)
        poff = pl.multiple_of((ebase + m * MAC) // 2, 8)
        ld_z = pltpu.async_copy(ez_o.at[pl.ds(poff, MPAIRS)], ezpb, semb)

        @pl.when(cid == 0)
        def _():
            pltpu.async_copy(ei1_h.at[1, pl.ds(loff, MAC)], dstb, semi).wait()

        @pl.when(cid == 1)
        def _():
            pltpu.async_copy(ei2_h.at[1, pl.ds(loff, MAC)], dstb, semi).wait()

        def chunk2(j, cc):
            def cpi2(k, cc2):
                dstscv[pl.ds(k * 16, 16)] = dstb[pl.ds(j * CH + k * 16, 16)]
                return cc2

            lax.fori_loop(0, CH // 16, cpi2, 0)
            g_d = pltpu.async_copy(den_sh.at[dstscv], erv, sema)
            g_d.wait()

            def pair2(p, cc2):
                d0 = erv[2 * p]
                d1 = erv[2 * p + 1]
                dp = jnp.where(lmask, d0, d1) + 1e-16
                r = j * PAIRS + p
                apb[r] = ezpb[r] / dp
                return cc2

            lax.fori_loop(0, PAIRS, pair2, 0)
            return cc

        ld_z.wait()
        lax.fori_loop(0, CPM, chunk2, 0)
        lpoff = pl.multiple_of(sid * (EPT // 2) + m * MPAIRS, 8)

        @pl.when(cid == 0)
        def _():
            pltpu.sync_copy(apb, alpha1_o.at[pl.ds(lpoff, MPAIRS)])

        @pl.when(cid == 1)
        def _():
            pltpu.sync_copy(apb, alpha2_o.at[pl.ds(lpoff, MPAIRS)])

        return c

    lax.fori_loop(0, NMC, macro2, 0)


@functools.cache
def _get_sc_call():
    return pl.kernel(
        _sc_body,
        out_type=(
            jax.ShapeDtypeStruct((2, NPAD, HID), jnp.float32),
            jax.ShapeDtypeStruct((E // 2, 16), jnp.float32),
            jax.ShapeDtypeStruct((E // 2, 16), jnp.float32),
            jax.ShapeDtypeStruct((E, 16), jnp.float32),
        ),
        mesh=plsc.VectorSubcoreMesh(core_axis_name="c", subcore_axis_name="s",
                                    num_cores=2, num_subcores=NS),
        compiler_params=pltpu.CompilerParams(use_tc_tiling_on_sc=False),
        scratch_types=[
            pltpu.VMEM_SHARED((NPAD, HID), jnp.float32),
            pltpu.VMEM_SHARED((NPAD, 16), jnp.float32),
            pltpu.VMEM((MAC,), jnp.int32),
            pltpu.VMEM((MAC,), jnp.int32),
            pltpu.VMEM((MAC,), jnp.int32),
            pltpu.VMEM((CH,), jnp.int32),
            pltpu.VMEM((CH,), jnp.int32),
            pltpu.VMEM((CH,), jnp.int32),
            pltpu.VMEM((CH, 16), jnp.float32),
            pltpu.VMEM((CH, 16), jnp.float32),
            pltpu.VMEM((CH, 16), jnp.float32),
            pltpu.VMEM((MPAIRS, 16), jnp.float32),
            pltpu.VMEM((MPAIRS, 16), jnp.float32),
            pltpu.VMEM((CH, HID), jnp.float32),
            pltpu.VMEM((NB, HID), jnp.float32),
            pltpu.VMEM((NB, 16), jnp.float32),
            pltpu.SemaphoreType.DMA,
            pltpu.SemaphoreType.DMA,
            pltpu.SemaphoreType.DMA,
            pltpu.SemaphoreType.DMA,
            pltpu.SemaphoreType.DMA,
            pltpu.SemaphoreType.DMA,
        ],
    )


def _mk_diag(a):
    rows = jnp.arange(HID)
    cols = rows // D_H
    z = jnp.zeros((HID, H), jnp.float32)
    return z.at[rows, cols].set(a.reshape(-1))


def kernel(x, edge_index_pap, edge_index_pfp, W_pap, al_pap, ar_pap, b_pap,
           W_pfp, al_pfp, ar_pfp, b_pfp, W_sem, b_sem, W_fc, b_fc):
    ALp, ARp = _mk_diag(al_pap), _mk_diag(ar_pap)
    ALf, ARf = _mk_diag(al_pfp), _mk_diag(ar_pfp)
    LW = jnp.stack([
        jnp.concatenate([ALp, ALp, ARp, ARp], axis=1),
        jnp.concatenate([ALf, ALf, ARf, ARf], axis=1),
    ])
    W_s = jnp.stack([W_pap, W_pfp])

    feat_s, el_s, er_s = _pre_call(x, W_s, LW)
    h_o, alpha1_o, alpha2_o, _ = _get_sc_call()(
        feat_s.reshape(2 * N, HID),
        el_s.reshape(2 * N, 16),
        er_s.reshape(2 * N, 16),
        edge_index_pap,
        edge_index_pfp,
    )
    out = _post_call(
        h_o, h_o,
        b_pap.reshape(1, HID), b_pfp.reshape(1, HID),
        W_sem.reshape(1, HID),
        W_fc, b_fc.reshape(1, D_OUT),
    )
    a1 = alpha1_o.reshape(E, H, 1)
    a2 = alpha2_o.reshape(E, H, 1)
    return out, a1, a2


# speedup vs baseline: 68.6751x; 1.0543x over previous
import functools

import jax
import jax.numpy as jnp
from jax import lax
from jax.experimental import pallas as pl
from jax.experimental.pallas import tpu as pltpu
from jax.experimental.pallas import tpu_sc as plsc

N = 10000
E = 320000
D_IN = 128
H = 8
D_H = 16
HID = H * D_H
D_OUT = 128
NEG = 0.2

NS = 16
EPT = E // NS
CH = 80
NCHUNK = EPT // CH
PAIRS = CH // 2
NPAD = 10240
NPT = NPAD // NS
NB = 64
NNB = NPT // NB
MAC = 400
NMC = EPT // MAC
CPM = MAC // CH
MPAIRS = MAC // 2

BN = 2000


def _pre_body(x_ref, w_ref, lw_ref, feat_ref, el_ref, er_ref):
    feat = jnp.dot(x_ref[...], w_ref[0], preferred_element_type=jnp.float32)
    lg = jnp.dot(feat, lw_ref[0], preferred_element_type=jnp.float32)
    feat_ref[...] = feat[None]
    el_ref[...] = lg[:, :16][None]
    er_ref[...] = lg[:, 16:][None]


_pre_call = pl.pallas_call(
    _pre_body,
    grid=(2, N // BN),
    in_specs=[
        pl.BlockSpec((BN, D_IN), lambda m, i: (i, 0)),
        pl.BlockSpec((1, D_IN, HID), lambda m, i: (m, 0, 0)),
        pl.BlockSpec((1, HID, 32), lambda m, i: (m, 0, 0)),
    ],
    out_specs=[
        pl.BlockSpec((1, BN, HID), lambda m, i: (m, i, 0)),
        pl.BlockSpec((1, BN, 16), lambda m, i: (m, i, 0)),
        pl.BlockSpec((1, BN, 16), lambda m, i: (m, i, 0)),
    ],
    out_shape=[
        jax.ShapeDtypeStruct((2, N, HID), jnp.float32),
        jax.ShapeDtypeStruct((2, N, 16), jnp.float32),
        jax.ShapeDtypeStruct((2, N, 16), jnp.float32),
    ],
)


def _post_body(h1_ref, h2_ref, b1_ref, b2_ref, ws_ref, wfc_ref, bfc_ref,
               out_ref):
    h1 = h1_ref[0] + b1_ref[...]
    h2 = h2_ref[0] + b2_ref[...]
    ws = ws_ref[...]
    s1 = jnp.sum(h1 * ws, axis=1, keepdims=True)
    s2 = jnp.sum(h2 * ws, axis=1, keepdims=True)
    m = jnp.maximum(s1, s2)
    e1 = jnp.exp(s1 - m)
    e2 = jnp.exp(s2 - m)
    h = (e1 * h1 + e2 * h2) / (e1 + e2)
    out_ref[...] = (
        jnp.dot(h, wfc_ref[...], preferred_element_type=jnp.float32)
        + bfc_ref[...]
    )


_post_call = pl.pallas_call(
    _post_body,
    grid=(N // BN,),
    in_specs=[
        pl.BlockSpec((1, BN, HID), lambda i: (0, i, 0)),
        pl.BlockSpec((1, BN, HID), lambda i: (1, i, 0)),
        pl.BlockSpec((1, HID), lambda i: (0, 0)),
        pl.BlockSpec((1, HID), lambda i: (0, 0)),
        pl.BlockSpec((1, HID), lambda i: (0, 0)),
        pl.BlockSpec((HID, D_OUT), lambda i: (0, 0)),
        pl.BlockSpec((1, D_OUT), lambda i: (0, 0)),
    ],
    out_specs=pl.BlockSpec((BN, D_OUT), lambda i: (i, 0)),
    out_shape=jax.ShapeDtypeStruct((N, D_OUT), jnp.float32),
)


def _sc_body(feat_h, el_h, er_h, ei1_h, ei2_h,
             h_o, alpha1_o, alpha2_o, ez_o,
             rst_sh, den_sh,
             srcb, dstb, dstab,
             srcsc0, srcsc1, dstsc0, dstsc1, dstav0, dstav1,
             elv0, elv1, erv0, erv1, ezv, ezpb, apb, featv,
             nbufv, dbufv,
             semi, sema0, sema1, semb0, semb1, semc, semd, sems):
    cid = lax.axis_index("c")
    sid = lax.axis_index("s")
    nbase = cid * N
    ebase = cid * E + sid * EPT
    zero16 = jnp.zeros((16,), jnp.float32)
    lanes = lax.iota(jnp.int32, 16)
    lmask = lanes < 8

    def zrow(j, c):
        for t in range(H):
            nbufv[j, pl.ds(t * 16, 16)] = zero16
        dbufv[j, pl.ds(0, 16)] = zero16
        return c

    lax.fori_loop(0, NB, zrow, 0)

    def zcp(k, c):
        roff = sid * NPT + k * NB
        pltpu.sync_copy(nbufv, rst_sh.at[pl.ds(roff, NB)])
        pltpu.sync_copy(dbufv, den_sh.at[pl.ds(roff, NB)])
        return c

    lax.fori_loop(0, NNB, zcp, 0)
    plsc.subcore_barrier()

    def macro1(m, c):
        loff = pl.multiple_of(sid * EPT + m * MAC, 8)

        @pl.when(cid == 0)
        def _():
            ld_s = pltpu.async_copy(ei1_h.at[0, pl.ds(loff, MAC)], srcb, semi)
            ld_d = pltpu.async_copy(ei1_h.at[1, pl.ds(loff, MAC)], dstb, semi)
            ld_s.wait()
            ld_d.wait()

        @pl.when(cid == 1)
        def _():
            ld_s = pltpu.async_copy(ei2_h.at[0, pl.ds(loff, MAC)], srcb, semi)
            ld_d = pltpu.async_copy(ei2_h.at[1, pl.ds(loff, MAC)], dstb, semi)
            ld_s.wait()
            ld_d.wait()

        def adj(k, cc):
            s = pl.ds(k * 16, 16)
            srcb[s] = srcb[s] + nbase
            dstab[s] = dstb[s] + nbase
            return cc

        lax.fori_loop(0, MAC // 16, adj, 0)

        srcsc = (srcsc0, srcsc1)
        dstsc = (dstsc0, dstsc1)
        dstav = (dstav0, dstav1)
        elv = (elv0, elv1)
        erv = (erv0, erv1)
        sema = (sema0, sema1)
        semb = (semb0, semb1)

        def cpi(j, b):
            def body(k, cc2):
                s = pl.ds(k * 16, 16)
                t = pl.ds(j * CH + k * 16, 16)
                srcsc[b][s] = srcb[t]
                dstsc[b][s] = dstb[t]
                dstav[b][s] = dstab[t]
                return cc2

            lax.fori_loop(0, CH // 16, body, 0)

        def fire_elr(b):
            pltpu.async_copy(el_h.at[srcsc[b]], elv[b], sema[b])
            pltpu.async_copy(er_h.at[dstav[b]], erv[b], semb[b])

        @pl.when(m > 0)
        def _():
            pltpu.make_async_copy(featv, rst_sh.at[dstsc0], sems).wait()

        cpi(0, 0)
        fire_elr(0)

        for j in range(CPM):
            b = j % 2
            if j > 0:
                pltpu.make_async_copy(featv, rst_sh.at[dstsc0], sems).wait()
            g_f = pltpu.async_copy(feat_h.at[srcsc[b]], featv, semc)
            if j + 1 < CPM:
                cpi(j + 1, 1 - b)
                fire_elr(1 - b)
            pltpu.make_async_copy(el_h.at[srcsc[b]], elv[b], sema[b]).wait()
            pltpu.make_async_copy(er_h.at[dstav[b]], erv[b], semb[b]).wait()

            def pair(p, cc2, _j=j, _b=b):
                e0 = elv[_b][2 * p] + erv[_b][2 * p]
                e0 = jnp.where(e0 >= 0, e0, NEG * e0)
                z0 = jnp.exp(e0)
                e1 = elv[_b][2 * p + 1] + erv[_b][2 * p + 1]
                e1 = jnp.where(e1 >= 0, e1, NEG * e1)
                z1 = jnp.exp(e1)
                ezv[2 * p] = z0
                ezv[2 * p + 1] = z1
                ezpb[_j * PAIRS + p] = jnp.where(lmask, z0, z1)
                return cc2

            lax.fori_loop(0, PAIRS, pair, 0)
            d_den = pltpu.async_copy(ezv, den_sh.at[dstsc[b]], semd, add=True)
            g_f.wait()

            def scale(q, cc2):
                ez_row = ezv[q]
                for t in range(H):
                    sl = pl.ds(t * 16, 16)
                    featv[q, sl] = featv[q, sl] * ez_row[t]
                return cc2

            lax.fori_loop(0, CH, scale, 0)
            d_den.wait()
            pltpu.async_copy(featv, rst_sh.at[dstsc[b]], sems,
                             add=True)
        poff = pl.multiple_of((ebase + m * MAC) // 2, 8)
        pltpu.sync_copy(ezpb, ez_o.at[pl.ds(poff, MPAIRS)])
        return c

    lax.fori_loop(0, NMC, macro1, 0)
    pltpu.make_async_copy(featv, rst_sh.at[dstsc0], sems).wait()
    plsc.subcore_barrier()

    def npass(k, c):
        roff = sid * NPT + k * NB
        pltpu.sync_copy(rst_sh.at[pl.ds(roff, NB)], nbufv)
        pltpu.sync_copy(den_sh.at[pl.ds(roff, NB)], dbufv)

        def nrow(j, cc):
            d_row = dbufv[j]
            for t in range(H):
                sl = pl.ds(t * 16, 16)
                nbufv[j, sl] = nbufv[j, sl] / (d_row[t] + 1e-16)
            return cc

        lax.fori_loop(0, NB, nrow, 0)
        pltpu.sync_copy(nbufv, h_o.at[cid, pl.ds(roff, NB)])
        return c

    lax.fori_loop(0, NNB, npass, 0)

    def macro2(m, c):
        loff = pl.multiple_of(sid * EPT + m * MAC, 8)
        poff = pl.multiple_of((ebase + m * MAC) // 2, 8)
        ld_z = pltpu.async_copy(ez_o.at[pl.ds(poff, MPAIRS)], ezpb, semb0)

        @pl.when(cid == 0)
        def _():
            pltpu.async_copy(ei1_h.at[1, pl.ds(loff, MAC)], dstb, semi).wait()

        @pl.when(cid == 1)
        def _():
            pltpu.async_copy(ei2_h.at[1, pl.ds(loff, MAC)], dstb, semi).wait()

        def chunk2(j, cc):
            def cpi2(k, cc2):
                dstsc0[pl.ds(k * 16, 16)] = dstb[pl.ds(j * CH + k * 16, 16)]
                return cc2

            lax.fori_loop(0, CH // 16, cpi2, 0)
            g_d = pltpu.async_copy(den_sh.at[dstsc0], erv0, sema0)
            g_d.wait()

            def pair2(p, cc2):
                d0 = erv0[2 * p]
                d1 = erv0[2 * p + 1]
                dp = jnp.where(lmask, d0, d1) + 1e-16
                r = j * PAIRS + p
                apb[r] = ezpb[r] / dp
                return cc2

            lax.fori_loop(0, PAIRS, pair2, 0)
            return cc

        ld_z.wait()
        lax.fori_loop(0, CPM, chunk2, 0)
        lpoff = pl.multiple_of(sid * (EPT // 2) + m * MPAIRS, 8)

        @pl.when(cid == 0)
        def _():
            pltpu.sync_copy(apb, alpha1_o.at[pl.ds(lpoff, MPAIRS)])

        @pl.when(cid == 1)
        def _():
            pltpu.sync_copy(apb, alpha2_o.at[pl.ds(lpoff, MPAIRS)])

        return c

    lax.fori_loop(0, NMC, macro2, 0)


@functools.cache
def _get_sc_call():
    return pl.kernel(
        _sc_body,
        out_type=(
            jax.ShapeDtypeStruct((2, NPAD, HID), jnp.float32),
            jax.ShapeDtypeStruct((E // 2, 16), jnp.float32),
            jax.ShapeDtypeStruct((E // 2, 16), jnp.float32),
            jax.ShapeDtypeStruct((E, 16), jnp.float32),
        ),
        mesh=plsc.VectorSubcoreMesh(core_axis_name="c", subcore_axis_name="s",
                                    num_cores=2, num_subcores=NS),
        compiler_params=pltpu.CompilerParams(use_tc_tiling_on_sc=False),
        scratch_types=[
            pltpu.VMEM_SHARED((NPAD, HID), jnp.float32),
            pltpu.VMEM_SHARED((NPAD, 16), jnp.float32),
            pltpu.VMEM((MAC,), jnp.int32),
            pltpu.VMEM((MAC,), jnp.int32),
            pltpu.VMEM((MAC,), jnp.int32),
            pltpu.VMEM((CH,), jnp.int32),
            pltpu.VMEM((CH,), jnp.int32),
            pltpu.VMEM((CH,), jnp.int32),
            pltpu.VMEM((CH,), jnp.int32),
            pltpu.VMEM((CH,), jnp.int32),
            pltpu.VMEM((CH,), jnp.int32),
            pltpu.VMEM((CH, 16), jnp.float32),
            pltpu.VMEM((CH, 16), jnp.float32),
            pltpu.VMEM((CH, 16), jnp.float32),
            pltpu.VMEM((CH, 16), jnp.float32),
            pltpu.VMEM((CH, 16), jnp.float32),
            pltpu.VMEM((MPAIRS, 16), jnp.float32),
            pltpu.VMEM((MPAIRS, 16), jnp.float32),
            pltpu.VMEM((CH, HID), jnp.float32),
            pltpu.VMEM((NB, HID), jnp.float32),
            pltpu.VMEM((NB, 16), jnp.float32),
            pltpu.SemaphoreType.DMA,
            pltpu.SemaphoreType.DMA,
            pltpu.SemaphoreType.DMA,
            pltpu.SemaphoreType.DMA,
            pltpu.SemaphoreType.DMA,
            pltpu.SemaphoreType.DMA,
            pltpu.SemaphoreType.DMA,
            pltpu.SemaphoreType.DMA,
        ],
    )


def _mk_diag(a):
    rows = jnp.arange(HID)
    cols = rows // D_H
    z = jnp.zeros((HID, H), jnp.float32)
    return z.at[rows, cols].set(a.reshape(-1))


def kernel(x, edge_index_pap, edge_index_pfp, W_pap, al_pap, ar_pap, b_pap,
           W_pfp, al_pfp, ar_pfp, b_pfp, W_sem, b_sem, W_fc, b_fc):
    ALp, ARp = _mk_diag(al_pap), _mk_diag(ar_pap)
    ALf, ARf = _mk_diag(al_pfp), _mk_diag(ar_pfp)
    LW = jnp.stack([
        jnp.concatenate([ALp, ALp, ARp, ARp], axis=1),
        jnp.concatenate([ALf, ALf, ARf, ARf], axis=1),
    ])
    W_s = jnp.stack([W_pap, W_pfp])

    feat_s, el_s, er_s = _pre_call(x, W_s, LW)
    h_o, alpha1_o, alpha2_o, _ = _get_sc_call()(
        feat_s.reshape(2 * N, HID),
        el_s.reshape(2 * N, 16),
        er_s.reshape(2 * N, 16),
        edge_index_pap,
        edge_index_pfp,
    )
    out = _post_call(
        h_o, h_o,
        b_pap.reshape(1, HID), b_pfp.reshape(1, HID),
        W_sem.reshape(1, HID),
        W_fc, b_fc.reshape(1, D_OUT),
    )
    a1 = alpha1_o.reshape(E, H, 1)
    a2 = alpha2_o.reshape(E, H, 1)
    return out, a1, a2


# speedup vs baseline: 71.4916x; 1.0410x over previous
import functools

import jax
import jax.numpy as jnp
from jax import lax
from jax.experimental import pallas as pl
from jax.experimental.pallas import tpu as pltpu
from jax.experimental.pallas import tpu_sc as plsc

N = 10000
E = 320000
D_IN = 128
H = 8
D_H = 16
HID = H * D_H
D_OUT = 128
NEG = 0.2

NS = 16
EPT = E // NS
CH = 80
NCHUNK = EPT // CH
PAIRS = CH // 2
NPAD = 10240
NPT = NPAD // NS
NB = 64
NNB = NPT // NB
MAC = 400
NMC = EPT // MAC
CPM = MAC // CH
MPAIRS = MAC // 2

BN = 2000


def _pre_body(x_ref, w_ref, lw_ref, feat_ref, el_ref, er_ref):
    feat = jnp.dot(x_ref[...], w_ref[0], preferred_element_type=jnp.float32)
    lg = jnp.dot(feat, lw_ref[0], preferred_element_type=jnp.float32)
    feat_ref[...] = feat[None]
    el_ref[...] = lg[:, :16][None]
    er_ref[...] = lg[:, 16:][None]


_pre_call = pl.pallas_call(
    _pre_body,
    grid=(2, N // BN),
    in_specs=[
        pl.BlockSpec((BN, D_IN), lambda m, i: (i, 0)),
        pl.BlockSpec((1, D_IN, HID), lambda m, i: (m, 0, 0)),
        pl.BlockSpec((1, HID, 32), lambda m, i: (m, 0, 0)),
    ],
    out_specs=[
        pl.BlockSpec((1, BN, HID), lambda m, i: (m, i, 0)),
        pl.BlockSpec((1, BN, 16), lambda m, i: (m, i, 0)),
        pl.BlockSpec((1, BN, 16), lambda m, i: (m, i, 0)),
    ],
    out_shape=[
        jax.ShapeDtypeStruct((2, N, HID), jnp.float32),
        jax.ShapeDtypeStruct((2, N, 16), jnp.float32),
        jax.ShapeDtypeStruct((2, N, 16), jnp.float32),
    ],
)


def _post_body(h1_ref, h2_ref, b1_ref, b2_ref, ws_ref, wfc_ref, bfc_ref,
               out_ref):
    h1 = h1_ref[0] + b1_ref[...]
    h2 = h2_ref[0] + b2_ref[...]
    ws = ws_ref[...]
    s1 = jnp.sum(h1 * ws, axis=1, keepdims=True)
    s2 = jnp.sum(h2 * ws, axis=1, keepdims=True)
    m = jnp.maximum(s1, s2)
    e1 = jnp.exp(s1 - m)
    e2 = jnp.exp(s2 - m)
    h = (e1 * h1 + e2 * h2) / (e1 + e2)
    out_ref[...] = (
        jnp.dot(h, wfc_ref[...], preferred_element_type=jnp.float32)
        + bfc_ref[...]
    )


_post_call = pl.pallas_call(
    _post_body,
    grid=(N // BN,),
    in_specs=[
        pl.BlockSpec((1, BN, HID), lambda i: (0, i, 0)),
        pl.BlockSpec((1, BN, HID), lambda i: (1, i, 0)),
        pl.BlockSpec((1, HID), lambda i: (0, 0)),
        pl.BlockSpec((1, HID), lambda i: (0, 0)),
        pl.BlockSpec((1, HID), lambda i: (0, 0)),
        pl.BlockSpec((HID, D_OUT), lambda i: (0, 0)),
        pl.BlockSpec((1, D_OUT), lambda i: (0, 0)),
    ],
    out_specs=pl.BlockSpec((BN, D_OUT), lambda i: (i, 0)),
    out_shape=jax.ShapeDtypeStruct((N, D_OUT), jnp.float32),
)


def _sc_body(feat_h, el_h, er_h, ei1_h, ei2_h,
             h_o, alpha1_o, alpha2_o, ez_o,
             rst_sh, den_sh,
             srcb, dstb, dstab,
             srcsc0, srcsc1, dstsc0, dstsc1, dstav0, dstav1,
             elv0, elv1, erv0, erv1, ezv, ezpb, apb, featv,
             nbufv, dbufv,
             semi, sema0, sema1, semb0, semb1, semc, semd, seme, sems):
    cid = lax.axis_index("c")
    sid = lax.axis_index("s")
    nbase = cid * N
    ebase = cid * E + sid * EPT
    zero16 = jnp.zeros((16,), jnp.float32)
    lanes = lax.iota(jnp.int32, 16)
    lmask = lanes < 8

    def zrow(j, c):
        for t in range(H):
            nbufv[j, pl.ds(t * 16, 16)] = zero16
        dbufv[j, pl.ds(0, 16)] = zero16
        return c

    lax.fori_loop(0, NB, zrow, 0)

    def zcp(k, c):
        roff = sid * NPT + k * NB
        pltpu.sync_copy(nbufv, rst_sh.at[pl.ds(roff, NB)])
        pltpu.sync_copy(dbufv, den_sh.at[pl.ds(roff, NB)])
        return c

    lax.fori_loop(0, NNB, zcp, 0)
    plsc.subcore_barrier()

    def macro1(m, c):
        loff = pl.multiple_of(sid * EPT + m * MAC, 8)

        @pl.when(cid == 0)
        def _():
            ld_s = pltpu.async_copy(ei1_h.at[0, pl.ds(loff, MAC)], srcb, semi)
            ld_d = pltpu.async_copy(ei1_h.at[1, pl.ds(loff, MAC)], dstb, semi)
            ld_s.wait()
            ld_d.wait()

        @pl.when(cid == 1)
        def _():
            ld_s = pltpu.async_copy(ei2_h.at[0, pl.ds(loff, MAC)], srcb, semi)
            ld_d = pltpu.async_copy(ei2_h.at[1, pl.ds(loff, MAC)], dstb, semi)
            ld_s.wait()
            ld_d.wait()

        def adj(k, cc):
            s = pl.ds(k * 16, 16)
            srcb[s] = srcb[s] + nbase
            dstab[s] = dstb[s] + nbase
            return cc

        lax.fori_loop(0, MAC // 16, adj, 0)
        @pl.when(m > 0)
        def _():
            pltpu.make_async_copy(ezpb, ez_o.at[pl.ds(0, MPAIRS)], seme).wait()

        srcsc = (srcsc0, srcsc1)
        dstsc = (dstsc0, dstsc1)
        dstav = (dstav0, dstav1)
        elv = (elv0, elv1)
        erv = (erv0, erv1)
        sema = (sema0, sema1)
        semb = (semb0, semb1)

        def cpi(j, b):
            def body(k, cc2):
                s = pl.ds(k * 16, 16)
                t = pl.ds(j * CH + k * 16, 16)
                srcsc[b][s] = srcb[t]
                dstsc[b][s] = dstb[t]
                dstav[b][s] = dstab[t]
                return cc2

            lax.fori_loop(0, CH // 16, body, 0)

        def fire_elr(b):
            pltpu.async_copy(el_h.at[srcsc[b]], elv[b], sema[b])
            pltpu.async_copy(er_h.at[dstav[b]], erv[b], semb[b])

        @pl.when(m > 0)
        def _():
            pltpu.make_async_copy(featv, rst_sh.at[dstsc0], sems).wait()

        cpi(0, 0)
        fire_elr(0)

        for j in range(CPM):
            b = j % 2
            if j > 0:
                pltpu.make_async_copy(featv, rst_sh.at[dstsc0], sems).wait()
            g_f = pltpu.async_copy(feat_h.at[srcsc[b]], featv, semc)
            if j + 1 < CPM:
                cpi(j + 1, 1 - b)
                fire_elr(1 - b)
            pltpu.make_async_copy(el_h.at[srcsc[b]], elv[b], sema[b]).wait()
            pltpu.make_async_copy(er_h.at[dstav[b]], erv[b], semb[b]).wait()

            def pair(p, cc2, _j=j, _b=b):
                e0 = elv[_b][2 * p] + erv[_b][2 * p]
                e0 = jnp.where(e0 >= 0, e0, NEG * e0)
                z0 = jnp.exp(e0)
                e1 = elv[_b][2 * p + 1] + erv[_b][2 * p + 1]
                e1 = jnp.where(e1 >= 0, e1, NEG * e1)
                z1 = jnp.exp(e1)
                ezv[2 * p] = z0
                ezv[2 * p + 1] = z1
                ezpb[_j * PAIRS + p] = jnp.where(lmask, z0, z1)
                return cc2

            lax.fori_loop(0, PAIRS, pair, 0)
            d_den = pltpu.async_copy(ezv, den_sh.at[dstsc[b]], semd, add=True)
            g_f.wait()

            def scale(q, cc2):
                ez_row = ezv[q]
                for t in range(H):
                    sl = pl.ds(t * 16, 16)
                    featv[q, sl] = featv[q, sl] * ez_row[t]
                return cc2

            lax.fori_loop(0, CH, scale, 0)
            d_den.wait()
            pltpu.async_copy(featv, rst_sh.at[dstsc[b]], sems,
                             add=True)
        poff = pl.multiple_of((ebase + m * MAC) // 2, 8)
        pltpu.async_copy(ezpb, ez_o.at[pl.ds(poff, MPAIRS)],
                         seme)
        return c

    lax.fori_loop(0, NMC, macro1, 0)
    pltpu.make_async_copy(featv, rst_sh.at[dstsc0], sems).wait()
    pltpu.make_async_copy(ezpb, ez_o.at[pl.ds(0, MPAIRS)], seme).wait()
    plsc.subcore_barrier()

    def npass(k, c):
        roff = sid * NPT + k * NB
        pltpu.sync_copy(rst_sh.at[pl.ds(roff, NB)], nbufv)
        pltpu.sync_copy(den_sh.at[pl.ds(roff, NB)], dbufv)

        def nrow(j, cc):
            d_row = dbufv[j]
            for t in range(H):
                sl = pl.ds(t * 16, 16)
                nbufv[j, sl] = nbufv[j, sl] / (d_row[t] + 1e-16)
            return cc

        lax.fori_loop(0, NB, nrow, 0)
        pltpu.sync_copy(nbufv, h_o.at[cid, pl.ds(roff, NB)])
        return c

    lax.fori_loop(0, NNB, npass, 0)

    def macro2(m, c):
        loff = pl.multiple_of(sid * EPT + m * MAC, 8)
        poff = pl.multiple_of((ebase + m * MAC) // 2, 8)
        ld_z = pltpu.async_copy(ez_o.at[pl.ds(poff, MPAIRS)], ezpb, semb0)

        @pl.when(cid == 0)
        def _():
            pltpu.async_copy(ei1_h.at[1, pl.ds(loff, MAC)], dstb, semi).wait()

        @pl.when(cid == 1)
        def _():
            pltpu.async_copy(ei2_h.at[1, pl.ds(loff, MAC)], dstb, semi).wait()

        dstsc = (dstsc0, dstsc1)
        erv = (erv0, erv1)
        sema = (sema0, sema1)

        def cpi2(j, b):
            def body(k, cc2):
                dstsc[b][pl.ds(k * 16, 16)] = dstb[pl.ds(j * CH + k * 16, 16)]
                return cc2

            lax.fori_loop(0, CH // 16, body, 0)

        cpi2(0, 0)
        pltpu.async_copy(den_sh.at[dstsc0], erv0, sema0)
        ld_z.wait()
        for j in range(CPM):
            b = j % 2
            if j + 1 < CPM:
                cpi2(j + 1, 1 - b)
                pltpu.async_copy(den_sh.at[dstsc[1 - b]], erv[1 - b],
                                 sema[1 - b])
            pltpu.make_async_copy(den_sh.at[dstsc[b]], erv[b],
                                  sema[b]).wait()

            def pair2(p, cc2, _j=j, _b=b):
                d0 = erv[_b][2 * p]
                d1 = erv[_b][2 * p + 1]
                dp = jnp.where(lmask, d0, d1) + 1e-16
                r = _j * PAIRS + p
                apb[r] = ezpb[r] / dp
                return cc2

            lax.fori_loop(0, PAIRS, pair2, 0)
        lpoff = pl.multiple_of(sid * (EPT // 2) + m * MPAIRS, 8)

        @pl.when(cid == 0)
        def _():
            pltpu.sync_copy(apb, alpha1_o.at[pl.ds(lpoff, MPAIRS)])

        @pl.when(cid == 1)
        def _():
            pltpu.sync_copy(apb, alpha2_o.at[pl.ds(lpoff, MPAIRS)])

        return c

    lax.fori_loop(0, NMC, macro2, 0)


@functools.cache
def _get_sc_call():
    return pl.kernel(
        _sc_body,
        out_type=(
            jax.ShapeDtypeStruct((2, NPAD, HID), jnp.float32),
            jax.ShapeDtypeStruct((E // 2, 16), jnp.float32),
            jax.ShapeDtypeStruct((E // 2, 16), jnp.float32),
            jax.ShapeDtypeStruct((E, 16), jnp.float32),
        ),
        mesh=plsc.VectorSubcoreMesh(core_axis_name="c", subcore_axis_name="s",
                                    num_cores=2, num_subcores=NS),
        compiler_params=pltpu.CompilerParams(use_tc_tiling_on_sc=False),
        scratch_types=[
            pltpu.VMEM_SHARED((NPAD, HID), jnp.float32),
            pltpu.VMEM_SHARED((NPAD, 16), jnp.float32),
            pltpu.VMEM((MAC,), jnp.int32),
            pltpu.VMEM((MAC,), jnp.int32),
            pltpu.VMEM((MAC,), jnp.int32),
            pltpu.VMEM((CH,), jnp.int32),
            pltpu.VMEM((CH,), jnp.int32),
            pltpu.VMEM((CH,), jnp.int32),
            pltpu.VMEM((CH,), jnp.int32),
            pltpu.VMEM((CH,), jnp.int32),
            pltpu.VMEM((CH,), jnp.int32),
            pltpu.VMEM((CH, 16), jnp.float32),
            pltpu.VMEM((CH, 16), jnp.float32),
            pltpu.VMEM((CH, 16), jnp.float32),
            pltpu.VMEM((CH, 16), jnp.float32),
            pltpu.VMEM((CH, 16), jnp.float32),
            pltpu.VMEM((MPAIRS, 16), jnp.float32),
            pltpu.VMEM((MPAIRS, 16), jnp.float32),
            pltpu.VMEM((CH, HID), jnp.float32),
            pltpu.VMEM((NB, HID), jnp.float32),
            pltpu.VMEM((NB, 16), jnp.float32),
            pltpu.SemaphoreType.DMA,
            pltpu.SemaphoreType.DMA,
            pltpu.SemaphoreType.DMA,
            pltpu.SemaphoreType.DMA,
            pltpu.SemaphoreType.DMA,
            pltpu.SemaphoreType.DMA,
            pltpu.SemaphoreType.DMA,
            pltpu.SemaphoreType.DMA,
            pltpu.SemaphoreType.DMA,
        ],
    )


def _mk_diag(a):
    rows = jnp.arange(HID)
    cols = rows // D_H
    z = jnp.zeros((HID, H), jnp.float32)
    return z.at[rows, cols].set(a.reshape(-1))


def kernel(x, edge_index_pap, edge_index_pfp, W_pap, al_pap, ar_pap, b_pap,
           W_pfp, al_pfp, ar_pfp, b_pfp, W_sem, b_sem, W_fc, b_fc):
    ALp, ARp = _mk_diag(al_pap), _mk_diag(ar_pap)
    ALf, ARf = _mk_diag(al_pfp), _mk_diag(ar_pfp)
    LW = jnp.stack([
        jnp.concatenate([ALp, ALp, ARp, ARp], axis=1),
        jnp.concatenate([ALf, ALf, ARf, ARf], axis=1),
    ])
    W_s = jnp.stack([W_pap, W_pfp])

    feat_s, el_s, er_s = _pre_call(x, W_s, LW)
    h_o, alpha1_o, alpha2_o, _ = _get_sc_call()(
        feat_s.reshape(2 * N, HID),
        el_s.reshape(2 * N, 16),
        er_s.reshape(2 * N, 16),
        edge_index_pap,
        edge_index_pfp,
    )
    out = _post_call(
        h_o, h_o,
        b_pap.reshape(1, HID), b_pfp.reshape(1, HID),
        W_sem.reshape(1, HID),
        W_fc, b_fc.reshape(1, D_OUT),
    )
    a1 = alpha1_o.reshape(E, H, 1)
    a2 = alpha2_o.reshape(E, H, 1)
    return out, a1, a2


# speedup vs baseline: 74.1947x; 1.0378x over previous
import functools

import jax
import jax.numpy as jnp
from jax import lax
from jax.experimental import pallas as pl
from jax.experimental.pallas import tpu as pltpu
from jax.experimental.pallas import tpu_sc as plsc

N = 10000
E = 320000
D_IN = 128
H = 8
D_H = 16
HID = H * D_H
D_OUT = 128
NEG = 0.2

NS = 16
EPT = E // NS
CH = 80
NCHUNK = EPT // CH
PAIRS = CH // 2
NPAD = 10240
NPT = NPAD // NS
NB = 80
NNB = NPT // NB
CHA = 48
CHB = CH - CHA
MAC = 400
NMC = EPT // MAC
CPM = MAC // CH
MPAIRS = MAC // 2

BN = 2000


def _pre_body(x_ref, w_ref, lw_ref, feat_ref, el_ref, er_ref):
    feat = jnp.dot(x_ref[...], w_ref[0], preferred_element_type=jnp.float32)
    lg = jnp.dot(feat, lw_ref[0], preferred_element_type=jnp.float32)
    feat_ref[...] = feat[None]
    el_ref[...] = lg[:, :16][None]
    er_ref[...] = lg[:, 16:][None]


_pre_call = pl.pallas_call(
    _pre_body,
    grid=(2, N // BN),
    in_specs=[
        pl.BlockSpec((BN, D_IN), lambda m, i: (i, 0)),
        pl.BlockSpec((1, D_IN, HID), lambda m, i: (m, 0, 0)),
        pl.BlockSpec((1, HID, 32), lambda m, i: (m, 0, 0)),
    ],
    out_specs=[
        pl.BlockSpec((1, BN, HID), lambda m, i: (m, i, 0)),
        pl.BlockSpec((1, BN, 16), lambda m, i: (m, i, 0)),
        pl.BlockSpec((1, BN, 16), lambda m, i: (m, i, 0)),
    ],
    out_shape=[
        jax.ShapeDtypeStruct((2, N, HID), jnp.float32),
        jax.ShapeDtypeStruct((2, N, 16), jnp.float32),
        jax.ShapeDtypeStruct((2, N, 16), jnp.float32),
    ],
)


def _post_body(h1_ref, h2_ref, b1_ref, b2_ref, ws_ref, wfc_ref, bfc_ref,
               out_ref):
    h1 = h1_ref[0] + b1_ref[...]
    h2 = h2_ref[0] + b2_ref[...]
    ws = ws_ref[...]
    s1 = jnp.sum(h1 * ws, axis=1, keepdims=True)
    s2 = jnp.sum(h2 * ws, axis=1, keepdims=True)
    m = jnp.maximum(s1, s2)
    e1 = jnp.exp(s1 - m)
    e2 = jnp.exp(s2 - m)
    h = (e1 * h1 + e2 * h2) / (e1 + e2)
    out_ref[...] = (
        jnp.dot(h, wfc_ref[...], preferred_element_type=jnp.float32)
        + bfc_ref[...]
    )


_post_call = pl.pallas_call(
    _post_body,
    grid=(N // BN,),
    in_specs=[
        pl.BlockSpec((1, BN, HID), lambda i: (0, i, 0)),
        pl.BlockSpec((1, BN, HID), lambda i: (1, i, 0)),
        pl.BlockSpec((1, HID), lambda i: (0, 0)),
        pl.BlockSpec((1, HID), lambda i: (0, 0)),
        pl.BlockSpec((1, HID), lambda i: (0, 0)),
        pl.BlockSpec((HID, D_OUT), lambda i: (0, 0)),
        pl.BlockSpec((1, D_OUT), lambda i: (0, 0)),
    ],
    out_specs=pl.BlockSpec((BN, D_OUT), lambda i: (i, 0)),
    out_shape=jax.ShapeDtypeStruct((N, D_OUT), jnp.float32),
)


def _sc_body(feat_h, el_h, er_h, ei1_h, ei2_h,
             h_o, alpha1_o, alpha2_o, ez_o,
             rst_sh, den_sh,
             srcb, dstb, dstab,
             srcsc0, srcsc1, dstsc0, dstsc1, dstav0, dstav1,
             dshA0, dshA1, dshB0, dshB1,
             elv0, elv1, erv0, erv1, ezv, ezpb, apb, featv,
             semi, sema0, sema1, semb0, semb1, semc, semd, seme, sems):
    cid = lax.axis_index("c")
    sid = lax.axis_index("s")
    nbase = cid * N
    ebase = cid * E + sid * EPT
    zero16 = jnp.zeros((16,), jnp.float32)
    lanes = lax.iota(jnp.int32, 16)
    lmask = lanes < 8

    def zrow(j, c):
        for t in range(H):
            featv[j, pl.ds(t * 16, 16)] = zero16
        elv0[j, pl.ds(0, 16)] = zero16
        return c

    lax.fori_loop(0, NB, zrow, 0)

    def zcp(k, c):
        roff = sid * NPT + k * NB
        pltpu.sync_copy(featv, rst_sh.at[pl.ds(roff, NB)])
        pltpu.sync_copy(elv0, den_sh.at[pl.ds(roff, NB)])
        return c

    lax.fori_loop(0, NNB, zcp, 0)
    plsc.subcore_barrier()

    def macro1(m, c):
        loff = pl.multiple_of(sid * EPT + m * MAC, 8)

        @pl.when(cid == 0)
        def _():
            ld_s = pltpu.async_copy(ei1_h.at[0, pl.ds(loff, MAC)], srcb, semi)
            ld_d = pltpu.async_copy(ei1_h.at[1, pl.ds(loff, MAC)], dstb, semi)
            ld_s.wait()
            ld_d.wait()

        @pl.when(cid == 1)
        def _():
            ld_s = pltpu.async_copy(ei2_h.at[0, pl.ds(loff, MAC)], srcb, semi)
            ld_d = pltpu.async_copy(ei2_h.at[1, pl.ds(loff, MAC)], dstb, semi)
            ld_s.wait()
            ld_d.wait()

        def adj(k, cc):
            s = pl.ds(k * 16, 16)
            srcb[s] = srcb[s] + nbase
            dstab[s] = dstb[s] + nbase
            return cc

        lax.fori_loop(0, MAC // 16, adj, 0)
        @pl.when(m > 0)
        def _():
            pltpu.make_async_copy(ezpb, ez_o.at[pl.ds(0, MPAIRS)], seme).wait()

        srcsc = (srcsc0, srcsc1)
        dstsc = (dstsc0, dstsc1)
        dstav = (dstav0, dstav1)
        dshA = (dshA0, dshA1)
        dshB = (dshB0, dshB1)
        elv = (elv0, elv1)
        erv = (erv0, erv1)
        sema = (sema0, sema1)
        semb = (semb0, semb1)

        def cpi(j, b):
            for k in range(CH // 16):
                s = pl.ds(k * 16, 16)
                t = pl.ds(j * CH + k * 16, 16)
                v = dstb[t]
                srcsc[b][s] = srcb[t]
                dstsc[b][s] = v
                dstav[b][s] = dstab[t]
                if k < CHA // 16:
                    dshA[b][s] = v
                else:
                    dshB[b][pl.ds(k * 16 - CHA, 16)] = v

        def fire_elr(b):
            pltpu.async_copy(el_h.at[srcsc[b]], elv[b], sema[b])
            pltpu.async_copy(er_h.at[dstav[b]], erv[b], semb[b])

        def drain_rst(b):
            pltpu.make_async_copy(featv.at[pl.ds(0, CHA)],
                                  rst_sh.at[dshA[b]], sems).wait()
            pltpu.make_async_copy(featv.at[pl.ds(CHA, CHB)],
                                  rst_sh.at[dshB[b]], sems).wait()

        @pl.when(m > 0)
        def _():
            drain_rst(0)

        cpi(0, 0)
        fire_elr(0)

        for j in range(CPM):
            b = j % 2
            if j > 0:
                drain_rst(1 - b)
            g_f = pltpu.async_copy(feat_h.at[srcsc[b]], featv, semc)
            if j + 1 < CPM:
                cpi(j + 1, 1 - b)
                fire_elr(1 - b)
            pltpu.make_async_copy(el_h.at[srcsc[b]], elv[b], sema[b]).wait()
            pltpu.make_async_copy(er_h.at[dstav[b]], erv[b], semb[b]).wait()

            def pair(p, cc2, _j=j, _b=b):
                e0 = elv[_b][2 * p] + erv[_b][2 * p]
                e0 = jnp.where(e0 >= 0, e0, NEG * e0)
                z0 = jnp.exp(e0)
                e1 = elv[_b][2 * p + 1] + erv[_b][2 * p + 1]
                e1 = jnp.where(e1 >= 0, e1, NEG * e1)
                z1 = jnp.exp(e1)
                ezv[2 * p] = z0
                ezv[2 * p + 1] = z1
                ezpb[_j * PAIRS + p] = jnp.where(lmask, z0, z1)
                return cc2

            lax.fori_loop(0, PAIRS, pair, 0)
            d_den = pltpu.async_copy(ezv, den_sh.at[dstsc[b]], semd, add=True)
            g_f.wait()

            def scale(q, cc2):
                ez_row = ezv[q]
                for t in range(H):
                    sl = pl.ds(t * 16, 16)
                    featv[q, sl] = featv[q, sl] * ez_row[t]
                return cc2

            lax.fori_loop(0, CHA, scale, 0)
            pltpu.async_copy(featv.at[pl.ds(0, CHA)], rst_sh.at[dshA[b]],
                             sems, add=True)
            lax.fori_loop(CHA, CH, scale, 0)
            d_den.wait()
            pltpu.async_copy(featv.at[pl.ds(CHA, CHB)], rst_sh.at[dshB[b]],
                             sems, add=True)
        poff = pl.multiple_of((ebase + m * MAC) // 2, 8)
        pltpu.async_copy(ezpb, ez_o.at[pl.ds(poff, MPAIRS)],
                         seme)
        return c

    lax.fori_loop(0, NMC, macro1, 0)
    pltpu.make_async_copy(featv, rst_sh.at[dstsc0], sems).wait()
    pltpu.make_async_copy(ezpb, ez_o.at[pl.ds(0, MPAIRS)], seme).wait()
    plsc.subcore_barrier()

    def npass(k, c):
        roff = sid * NPT + k * NB
        pltpu.sync_copy(rst_sh.at[pl.ds(roff, NB)], featv)
        pltpu.sync_copy(den_sh.at[pl.ds(roff, NB)], elv0)

        def nrow(j, cc):
            d_row = elv0[j]
            for t in range(H):
                sl = pl.ds(t * 16, 16)
                featv[j, sl] = featv[j, sl] / (d_row[t] + 1e-16)
            return cc

        lax.fori_loop(0, NB, nrow, 0)
        pltpu.sync_copy(featv, h_o.at[cid, pl.ds(roff, NB)])
        return c

    lax.fori_loop(0, NNB, npass, 0)

    def macro2(m, c):
        loff = pl.multiple_of(sid * EPT + m * MAC, 8)
        poff = pl.multiple_of((ebase + m * MAC) // 2, 8)
        ld_z = pltpu.async_copy(ez_o.at[pl.ds(poff, MPAIRS)], ezpb, semb0)

        @pl.when(cid == 0)
        def _():
            pltpu.async_copy(ei1_h.at[1, pl.ds(loff, MAC)], dstb, semi).wait()

        @pl.when(cid == 1)
        def _():
            pltpu.async_copy(ei2_h.at[1, pl.ds(loff, MAC)], dstb, semi).wait()

        dstsc = (dstsc0, dstsc1)
        erv = (erv0, erv1)
        sema = (sema0, sema1)

        def cpi2(j, b):
            def body(k, cc2):
                dstsc[b][pl.ds(k * 16, 16)] = dstb[pl.ds(j * CH + k * 16, 16)]
                return cc2

            lax.fori_loop(0, CH // 16, body, 0)

        cpi2(0, 0)
        pltpu.async_copy(den_sh.at[dstsc0], erv0, sema0)
        ld_z.wait()
        for j in range(CPM):
            b = j % 2
            if j + 1 < CPM:
                cpi2(j + 1, 1 - b)
                pltpu.async_copy(den_sh.at[dstsc[1 - b]], erv[1 - b],
                                 sema[1 - b])
            pltpu.make_async_copy(den_sh.at[dstsc[b]], erv[b],
                                  sema[b]).wait()

            def pair2(p, cc2, _j=j, _b=b):
                d0 = erv[_b][2 * p]
                d1 = erv[_b][2 * p + 1]
                dp = jnp.where(lmask, d0, d1) + 1e-16
                r = _j * PAIRS + p
                apb[r] = ezpb[r] / dp
                return cc2

            lax.fori_loop(0, PAIRS, pair2, 0)
        lpoff = pl.multiple_of(sid * (EPT // 2) + m * MPAIRS, 8)

        @pl.when(cid == 0)
        def _():
            pltpu.sync_copy(apb, alpha1_o.at[pl.ds(lpoff, MPAIRS)])

        @pl.when(cid == 1)
        def _():
            pltpu.sync_copy(apb, alpha2_o.at[pl.ds(lpoff, MPAIRS)])

        return c

    lax.fori_loop(0, NMC, macro2, 0)


@functools.cache
def _get_sc_call():
    return pl.kernel(
        _sc_body,
        out_type=(
            jax.ShapeDtypeStruct((2, NPAD, HID), jnp.float32),
            jax.ShapeDtypeStruct((E // 2, 16), jnp.float32),
            jax.ShapeDtypeStruct((E // 2, 16), jnp.float32),
            jax.ShapeDtypeStruct((E, 16), jnp.float32),
        ),
        mesh=plsc.VectorSubcoreMesh(core_axis_name="c", subcore_axis_name="s",
                                    num_cores=2, num_subcores=NS),
        compiler_params=pltpu.CompilerParams(use_tc_tiling_on_sc=False),
        scratch_types=[
            pltpu.VMEM_SHARED((NPAD, HID), jnp.float32),
            pltpu.VMEM_SHARED((NPAD, 16), jnp.float32),
            pltpu.VMEM((MAC,), jnp.int32),
            pltpu.VMEM((MAC,), jnp.int32),
            pltpu.VMEM((MAC,), jnp.int32),
            pltpu.VMEM((CH,), jnp.int32),
            pltpu.VMEM((CH,), jnp.int32),
            pltpu.VMEM((CH,), jnp.int32),
            pltpu.VMEM((CH,), jnp.int32),
            pltpu.VMEM((CH,), jnp.int32),
            pltpu.VMEM((CH,), jnp.int32),
            pltpu.VMEM((CHA,), jnp.int32),
            pltpu.VMEM((CHA,), jnp.int32),
            pltpu.VMEM((CHB,), jnp.int32),
            pltpu.VMEM((CHB,), jnp.int32),
            pltpu.VMEM((CH, 16), jnp.float32),
            pltpu.VMEM((CH, 16), jnp.float32),
            pltpu.VMEM((CH, 16), jnp.float32),
            pltpu.VMEM((CH, 16), jnp.float32),
            pltpu.VMEM((CH, 16), jnp.float32),
            pltpu.VMEM((MPAIRS, 16), jnp.float32),
            pltpu.VMEM((MPAIRS, 16), jnp.float32),
            pltpu.VMEM((CH, HID), jnp.float32),
            pltpu.SemaphoreType.DMA,
            pltpu.SemaphoreType.DMA,
            pltpu.SemaphoreType.DMA,
            pltpu.SemaphoreType.DMA,
            pltpu.SemaphoreType.DMA,
            pltpu.SemaphoreType.DMA,
            pltpu.SemaphoreType.DMA,
            pltpu.SemaphoreType.DMA,
            pltpu.SemaphoreType.DMA,
        ],
    )


def _mk_diag(a):
    rows = jnp.arange(HID)
    cols = rows // D_H
    z = jnp.zeros((HID, H), jnp.float32)
    return z.at[rows, cols].set(a.reshape(-1))


def kernel(x, edge_index_pap, edge_index_pfp, W_pap, al_pap, ar_pap, b_pap,
           W_pfp, al_pfp, ar_pfp, b_pfp, W_sem, b_sem, W_fc, b_fc):
    ALp, ARp = _mk_diag(al_pap), _mk_diag(ar_pap)
    ALf, ARf = _mk_diag(al_pfp), _mk_diag(ar_pfp)
    LW = jnp.stack([
        jnp.concatenate([ALp, ALp, ARp, ARp], axis=1),
        jnp.concatenate([ALf, ALf, ARf, ARf], axis=1),
    ])
    W_s = jnp.stack([W_pap, W_pfp])

    feat_s, el_s, er_s = _pre_call(x, W_s, LW)
    h_o, alpha1_o, alpha2_o, _ = _get_sc_call()(
        feat_s.reshape(2 * N, HID),
        el_s.reshape(2 * N, 16),
        er_s.reshape(2 * N, 16),
        edge_index_pap,
        edge_index_pfp,
    )
    out = _post_call(
        h_o, h_o,
        b_pap.reshape(1, HID), b_pfp.reshape(1, HID),
        W_sem.reshape(1, HID),
        W_fc, b_fc.reshape(1, D_OUT),
    )
    a1 = alpha1_o.reshape(E, H, 1)
    a2 = alpha2_o.reshape(E, H, 1)
    return out, a1, a2


# speedup vs baseline: 75.9428x; 1.0236x over previous
import functools

import jax
import jax.numpy as jnp
from jax import lax
from jax.experimental import pallas as pl
from jax.experimental.pallas import tpu as pltpu
from jax.experimental.pallas import tpu_sc as plsc

N = 10000
E = 320000
D_IN = 128
H = 8
D_H = 16
HID = H * D_H
D_OUT = 128
NEG = 0.2

NS = 16
EPT = E // NS
CH = 80
NCHUNK = EPT // CH
PAIRS = CH // 2
NPAD = 10240
NPT = NPAD // NS
NB = 80
NNB = NPT // NB
CHA = 48
CHB = CH - CHA
MAC = 800
NMC = EPT // MAC
CPM = MAC // CH
MPAIRS = MAC // 2

BN = 2000


def _pre_body(x_ref, w_ref, lw_ref, feat_ref, el_ref, er_ref):
    feat = jnp.dot(x_ref[...], w_ref[0], preferred_element_type=jnp.float32)
    lg = jnp.dot(feat, lw_ref[0], preferred_element_type=jnp.float32)
    feat_ref[...] = feat[None]
    el_ref[...] = lg[:, :16][None]
    er_ref[...] = lg[:, 16:][None]


_pre_call = pl.pallas_call(
    _pre_body,
    grid=(2, N // BN),
    in_specs=[
        pl.BlockSpec((BN, D_IN), lambda m, i: (i, 0)),
        pl.BlockSpec((1, D_IN, HID), lambda m, i: (m, 0, 0)),
        pl.BlockSpec((1, HID, 32), lambda m, i: (m, 0, 0)),
    ],
    out_specs=[
        pl.BlockSpec((1, BN, HID), lambda m, i: (m, i, 0)),
        pl.BlockSpec((1, BN, 16), lambda m, i: (m, i, 0)),
        pl.BlockSpec((1, BN, 16), lambda m, i: (m, i, 0)),
    ],
    out_shape=[
        jax.ShapeDtypeStruct((2, N, HID), jnp.float32),
        jax.ShapeDtypeStruct((2, N, 16), jnp.float32),
        jax.ShapeDtypeStruct((2, N, 16), jnp.float32),
    ],
)


def _post_body(h1_ref, h2_ref, b1_ref, b2_ref, ws_ref, wfc_ref, bfc_ref,
               out_ref):
    h1 = h1_ref[0] + b1_ref[...]
    h2 = h2_ref[0] + b2_ref[...]
    ws = ws_ref[...]
    s1 = jnp.sum(h1 * ws, axis=1, keepdims=True)
    s2 = jnp.sum(h2 * ws, axis=1, keepdims=True)
    m = jnp.maximum(s1, s2)
    e1 = jnp.exp(s1 - m)
    e2 = jnp.exp(s2 - m)
    h = (e1 * h1 + e2 * h2) / (e1 + e2)
    out_ref[...] = (
        jnp.dot(h, wfc_ref[...], preferred_element_type=jnp.float32)
        + bfc_ref[...]
    )


_post_call = pl.pallas_call(
    _post_body,
    grid=(N // BN,),
    in_specs=[
        pl.BlockSpec((1, BN, HID), lambda i: (0, i, 0)),
        pl.BlockSpec((1, BN, HID), lambda i: (1, i, 0)),
        pl.BlockSpec((1, HID), lambda i: (0, 0)),
        pl.BlockSpec((1, HID), lambda i: (0, 0)),
        pl.BlockSpec((1, HID), lambda i: (0, 0)),
        pl.BlockSpec((HID, D_OUT), lambda i: (0, 0)),
        pl.BlockSpec((1, D_OUT), lambda i: (0, 0)),
    ],
    out_specs=pl.BlockSpec((BN, D_OUT), lambda i: (i, 0)),
    out_shape=jax.ShapeDtypeStruct((N, D_OUT), jnp.float32),
)


def _sc_body(feat_h, el_h, er_h, ei1_h, ei2_h,
             h_o, alpha1_o, alpha2_o, ez_o,
             rst_sh, den_sh,
             srcb, dstb, dstab,
             srcsc0, srcsc1, dstsc0, dstsc1, dstav0, dstav1,
             dshA0, dshA1, dshB0, dshB1,
             elv0, elv1, erv0, erv1, ezv, ezpb, apb, featv,
             semi, sema0, sema1, semb0, semb1, semc, semd, seme, sems):
    cid = lax.axis_index("c")
    sid = lax.axis_index("s")
    nbase = cid * N
    ebase = cid * E + sid * EPT
    zero16 = jnp.zeros((16,), jnp.float32)
    lanes = lax.iota(jnp.int32, 16)
    lmask = lanes < 8

    def zrow(j, c):
        for t in range(H):
            featv[j, pl.ds(t * 16, 16)] = zero16
        elv0[j, pl.ds(0, 16)] = zero16
        return c

    lax.fori_loop(0, NB, zrow, 0)

    def zcp(k, c):
        roff = sid * NPT + k * NB
        pltpu.sync_copy(featv, rst_sh.at[pl.ds(roff, NB)])
        pltpu.sync_copy(elv0, den_sh.at[pl.ds(roff, NB)])
        return c

    lax.fori_loop(0, NNB, zcp, 0)
    plsc.subcore_barrier()

    def macro1(m, c):
        loff = pl.multiple_of(sid * EPT + m * MAC, 8)

        @pl.when(cid == 0)
        def _():
            ld_s = pltpu.async_copy(ei1_h.at[0, pl.ds(loff, MAC)], srcb, semi)
            ld_d = pltpu.async_copy(ei1_h.at[1, pl.ds(loff, MAC)], dstb, semi)
            ld_s.wait()
            ld_d.wait()

        @pl.when(cid == 1)
        def _():
            ld_s = pltpu.async_copy(ei2_h.at[0, pl.ds(loff, MAC)], srcb, semi)
            ld_d = pltpu.async_copy(ei2_h.at[1, pl.ds(loff, MAC)], dstb, semi)
            ld_s.wait()
            ld_d.wait()

        def adj(k, cc):
            s = pl.ds(k * 16, 16)
            srcb[s] = srcb[s] + nbase
            dstab[s] = dstb[s] + nbase
            return cc

        lax.fori_loop(0, MAC // 16, adj, 0)
        @pl.when(m > 0)
        def _():
            pltpu.make_async_copy(ezpb, ez_o.at[pl.ds(0, MPAIRS)], seme).wait()

        srcsc = (srcsc0, srcsc1)
        dstsc = (dstsc0, dstsc1)
        dstav = (dstav0, dstav1)
        dshA = (dshA0, dshA1)
        dshB = (dshB0, dshB1)
        elv = (elv0, elv1)
        erv = (erv0, erv1)
        sema = (sema0, sema1)
        semb = (semb0, semb1)

        def cpi(j, b):
            for k in range(CH // 16):
                s = pl.ds(k * 16, 16)
                t = pl.ds(j * CH + k * 16, 16)
                v = dstb[t]
                srcsc[b][s] = srcb[t]
                dstsc[b][s] = v
                dstav[b][s] = dstab[t]
                if k < CHA // 16:
                    dshA[b][s] = v
                else:
                    dshB[b][pl.ds(k * 16 - CHA, 16)] = v

        def fire_elr(b):
            pltpu.async_copy(el_h.at[srcsc[b]], elv[b], sema[b])
            pltpu.async_copy(er_h.at[dstav[b]], erv[b], semb[b])

        def drain_rst(b):
            pltpu.make_async_copy(featv.at[pl.ds(0, CHA)],
                                  rst_sh.at[dshA[b]], sems).wait()
            pltpu.make_async_copy(featv.at[pl.ds(CHA, CHB)],
                                  rst_sh.at[dshB[b]], sems).wait()

        @pl.when(m > 0)
        def _():
            drain_rst(0)

        cpi(0, 0)
        fire_elr(0)

        for j in range(CPM):
            b = j % 2
            if j > 0:
                drain_rst(1 - b)
            g_f = pltpu.async_copy(feat_h.at[srcsc[b]], featv, semc)
            if j + 1 < CPM:
                cpi(j + 1, 1 - b)
                fire_elr(1 - b)
            pltpu.make_async_copy(el_h.at[srcsc[b]], elv[b], sema[b]).wait()
            pltpu.make_async_copy(er_h.at[dstav[b]], erv[b], semb[b]).wait()

            def pair(p, cc2, _j=j, _b=b):
                e0 = elv[_b][2 * p] + erv[_b][2 * p]
                e0 = jnp.where(e0 >= 0, e0, NEG * e0)
                z0 = jnp.exp(e0)
                e1 = elv[_b][2 * p + 1] + erv[_b][2 * p + 1]
                e1 = jnp.where(e1 >= 0, e1, NEG * e1)
                z1 = jnp.exp(e1)
                ezv[2 * p] = z0
                ezv[2 * p + 1] = z1
                ezpb[_j * PAIRS + p] = jnp.where(lmask, z0, z1)
                return cc2

            lax.fori_loop(0, PAIRS, pair, 0)
            d_den = pltpu.async_copy(ezv, den_sh.at[dstsc[b]], semd, add=True)
            g_f.wait()

            def scale(q, cc2):
                ez_row = ezv[q]
                for t in range(H):
                    sl = pl.ds(t * 16, 16)
                    featv[q, sl] = featv[q, sl] * ez_row[t]
                return cc2

            lax.fori_loop(0, CHA, scale, 0)
            pltpu.async_copy(featv.at[pl.ds(0, CHA)], rst_sh.at[dshA[b]],
                             sems, add=True)
            lax.fori_loop(CHA, CH, scale, 0)
            d_den.wait()
            pltpu.async_copy(featv.at[pl.ds(CHA, CHB)], rst_sh.at[dshB[b]],
                             sems, add=True)
        poff = pl.multiple_of((ebase + m * MAC) // 2, 8)
        pltpu.async_copy(ezpb, ez_o.at[pl.ds(poff, MPAIRS)],
                         seme)
        return c

    lax.fori_loop(0, NMC, macro1, 0)
    pltpu.make_async_copy(featv, rst_sh.at[dstsc0], sems).wait()
    pltpu.make_async_copy(ezpb, ez_o.at[pl.ds(0, MPAIRS)], seme).wait()
    plsc.subcore_barrier()

    def npass(k, c):
        roff = sid * NPT + k * NB
        pltpu.sync_copy(rst_sh.at[pl.ds(roff, NB)], featv)
        pltpu.sync_copy(den_sh.at[pl.ds(roff, NB)], elv0)

        def nrow(j, cc):
            d_row = elv0[j]
            for t in range(H):
                sl = pl.ds(t * 16, 16)
                featv[j, sl] = featv[j, sl] / (d_row[t] + 1e-16)
            return cc

        lax.fori_loop(0, NB, nrow, 0)
        pltpu.sync_copy(featv, h_o.at[cid, pl.ds(roff, NB)])
        return c

    lax.fori_loop(0, NNB, npass, 0)

    def macro2(m, c):
        loff = pl.multiple_of(sid * EPT + m * MAC, 8)
        poff = pl.multiple_of((ebase + m * MAC) // 2, 8)
        ld_z = pltpu.async_copy(ez_o.at[pl.ds(poff, MPAIRS)], ezpb, semb0)

        @pl.when(cid == 0)
        def _():
            pltpu.async_copy(ei1_h.at[1, pl.ds(loff, MAC)], dstb, semi).wait()

        @pl.when(cid == 1)
        def _():
            pltpu.async_copy(ei2_h.at[1, pl.ds(loff, MAC)], dstb, semi).wait()

        dstsc = (dstsc0, dstsc1)
        erv = (erv0, erv1)
        sema = (sema0, sema1)

        def cpi2(j, b):
            def body(k, cc2):
                dstsc[b][pl.ds(k * 16, 16)] = dstb[pl.ds(j * CH + k * 16, 16)]
                return cc2

            lax.fori_loop(0, CH // 16, body, 0)

        cpi2(0, 0)
        pltpu.async_copy(den_sh.at[dstsc0], erv0, sema0)
        ld_z.wait()
        for j in range(CPM):
            b = j % 2
            if j + 1 < CPM:
                cpi2(j + 1, 1 - b)
                pltpu.async_copy(den_sh.at[dstsc[1 - b]], erv[1 - b],
                                 sema[1 - b])
            pltpu.make_async_copy(den_sh.at[dstsc[b]], erv[b],
                                  sema[b]).wait()

            def pair2(p, cc2, _j=j, _b=b):
                d0 = erv[_b][2 * p]
                d1 = erv[_b][2 * p + 1]
                dp = jnp.where(lmask, d0, d1) + 1e-16
                r = _j * PAIRS + p
                apb[r] = ezpb[r] / dp
                return cc2

            lax.fori_loop(0, PAIRS, pair2, 0)
        lpoff = pl.multiple_of(sid * (EPT // 2) + m * MPAIRS, 8)

        @pl.when(cid == 0)
        def _():
            pltpu.sync_copy(apb, alpha1_o.at[pl.ds(lpoff, MPAIRS)])

        @pl.when(cid == 1)
        def _():
            pltpu.sync_copy(apb, alpha2_o.at[pl.ds(lpoff, MPAIRS)])

        return c

    lax.fori_loop(0, NMC, macro2, 0)


@functools.cache
def _get_sc_call():
    return pl.kernel(
        _sc_body,
        out_type=(
            jax.ShapeDtypeStruct((2, NPAD, HID), jnp.float32),
            jax.ShapeDtypeStruct((E // 2, 16), jnp.float32),
            jax.ShapeDtypeStruct((E // 2, 16), jnp.float32),
            jax.ShapeDtypeStruct((E, 16), jnp.float32),
        ),
        mesh=plsc.VectorSubcoreMesh(core_axis_name="c", subcore_axis_name="s",
                                    num_cores=2, num_subcores=NS),
        compiler_params=pltpu.CompilerParams(use_tc_tiling_on_sc=False),
        scratch_types=[
            pltpu.VMEM_SHARED((NPAD, HID), jnp.float32),
            pltpu.VMEM_SHARED((NPAD, 16), jnp.float32),
            pltpu.VMEM((MAC,), jnp.int32),
            pltpu.VMEM((MAC,), jnp.int32),
            pltpu.VMEM((MAC,), jnp.int32),
            pltpu.VMEM((CH,), jnp.int32),
            pltpu.VMEM((CH,), jnp.int32),
            pltpu.VMEM((CH,), jnp.int32),
            pltpu.VMEM((CH,), jnp.int32),
            pltpu.VMEM((CH,), jnp.int32),
            pltpu.VMEM((CH,), jnp.int32),
            pltpu.VMEM((CHA,), jnp.int32),
            pltpu.VMEM((CHA,), jnp.int32),
            pltpu.VMEM((CHB,), jnp.int32),
            pltpu.VMEM((CHB,), jnp.int32),
            pltpu.VMEM((CH, 16), jnp.float32),
            pltpu.VMEM((CH, 16), jnp.float32),
            pltpu.VMEM((CH, 16), jnp.float32),
            pltpu.VMEM((CH, 16), jnp.float32),
            pltpu.VMEM((CH, 16), jnp.float32),
            pltpu.VMEM((MPAIRS, 16), jnp.float32),
            pltpu.VMEM((MPAIRS, 16), jnp.float32),
            pltpu.VMEM((CH, HID), jnp.float32),
            pltpu.SemaphoreType.DMA,
            pltpu.SemaphoreType.DMA,
            pltpu.SemaphoreType.DMA,
            pltpu.SemaphoreType.DMA,
            pltpu.SemaphoreType.DMA,
            pltpu.SemaphoreType.DMA,
            pltpu.SemaphoreType.DMA,
            pltpu.SemaphoreType.DMA,
            pltpu.SemaphoreType.DMA,
        ],
    )


def _mk_diag(a):
    rows = jnp.arange(HID)
    cols = rows // D_H
    z = jnp.zeros((HID, H), jnp.float32)
    return z.at[rows, cols].set(a.reshape(-1))


def kernel(x, edge_index_pap, edge_index_pfp, W_pap, al_pap, ar_pap, b_pap,
           W_pfp, al_pfp, ar_pfp, b_pfp, W_sem, b_sem, W_fc, b_fc):
    ALp, ARp = _mk_diag(al_pap), _mk_diag(ar_pap)
    ALf, ARf = _mk_diag(al_pfp), _mk_diag(ar_pfp)
    LW = jnp.stack([
        jnp.concatenate([ALp, ALp, ARp, ARp], axis=1),
        jnp.concatenate([ALf, ALf, ARf, ARf], axis=1),
    ])
    W_s = jnp.stack([W_pap, W_pfp])

    feat_s, el_s, er_s = _pre_call(x, W_s, LW)
    h_o, alpha1_o, alpha2_o, _ = _get_sc_call()(
        feat_s.reshape(2 * N, HID),
        el_s.reshape(2 * N, 16),
        er_s.reshape(2 * N, 16),
        edge_index_pap,
        edge_index_pfp,
    )
    out = _post_call(
        h_o, h_o,
        b_pap.reshape(1, HID), b_pfp.reshape(1, HID),
        W_sem.reshape(1, HID),
        W_fc, b_fc.reshape(1, D_OUT),
    )
    a1 = alpha1_o.reshape(E, H, 1)
    a2 = alpha2_o.reshape(E, H, 1)
    return out, a1, a2
